# baseline (speedup 1.0000x reference)
import functools

import jax
import jax.numpy as jnp
from jax import lax
from jax.experimental import pallas as pl
from jax.experimental.pallas import tpu as pltpu
from jax.experimental.pallas import tpu_sc as plsc

T = 2048
H = 1024
E = 64
K = 2
I = 64
A = T * K
BM = 64
BUF = 8192
NB = BUF // BM
NW = 32
TB = 256


def _gating_body(x_ref, wg_ref, i0_ref, i1_ref, w0_ref, w1_ref):
    xb = x_ref[...]
    logits = jnp.dot(xb, wg_ref[...], preferred_element_type=jnp.float32)
    m = jnp.max(logits, axis=-1, keepdims=True)
    ex = jnp.exp(logits - m)
    raw = ex / jnp.sum(ex, axis=-1, keepdims=True)
    lane = lax.broadcasted_iota(jnp.int32, raw.shape, 1)
    p1 = jnp.max(raw, axis=-1, keepdims=True)
    a1 = jnp.min(jnp.where(raw == p1, lane, E), axis=-1, keepdims=True)
    masked = jnp.where(lane == a1, -1.0, raw)
    p2 = jnp.max(masked, axis=-1, keepdims=True)
    a2 = jnp.min(jnp.where(masked == p2, lane, E), axis=-1, keepdims=True)
    e2 = jnp.exp(p2 - p1)
    w0 = 1.0 / (1.0 + e2)
    i0_ref[...] = a1.astype(jnp.int32)
    i1_ref[...] = a2.astype(jnp.int32)
    w0_ref[...] = jnp.broadcast_to(w0, (TB, 16))
    w1_ref[...] = jnp.broadcast_to(1.0 - w0, (TB, 16))


def _gating(xf, w_gate):
    col_i = jax.ShapeDtypeStruct((T, 1), jnp.int32)
    spl_f = jax.ShapeDtypeStruct((T, 16), jnp.float32)
    return pl.pallas_call(
        _gating_body,
        grid=(T // TB,),
        in_specs=[
            pl.BlockSpec((TB, H), lambda i: (i, 0)),
            pl.BlockSpec((H, E), lambda i: (0, 0)),
        ],
        out_specs=[
            pl.BlockSpec((TB, 1), lambda i: (i, 0)),
            pl.BlockSpec((TB, 1), lambda i: (i, 0)),
            pl.BlockSpec((TB, 16), lambda i: (i, 0)),
            pl.BlockSpec((TB, 16), lambda i: (i, 0)),
        ],
        out_shape=[col_i, col_i, spl_f, spl_f],
    )(xf, w_gate)


def _routing_body(idx_ref, pos_ref, be_ref):
    lane = lax.broadcasted_iota(jnp.int32, (TB, E), 1)
    nch = A // TB
    ohs = []
    run = jnp.zeros((1, E), jnp.float32)
    coffs = []
    for c in range(nch):
        ohc = (idx_ref[pl.ds(c * TB, TB), :] == lane).astype(jnp.float32)
        ohs.append(ohc)
        coffs.append(run)
        run = run + jnp.sum(ohc, axis=0, keepdims=True)
    counts = run
    padded = jnp.ceil(counts / BM) * BM
    li = lax.broadcasted_iota(jnp.int32, (E, E), 0)
    lj = lax.broadcasted_iota(jnp.int32, (E, E), 1)
    ustrict = (li < lj).astype(jnp.float32)
    offs = jnp.dot(padded, ustrict, preferred_element_type=jnp.float32)

    bstart = (lax.broadcasted_iota(jnp.int32, (NB, E), 0) * BM).astype(jnp.float32)
    lane_e = lax.broadcasted_iota(jnp.int32, (NB, E), 1)
    sel = (bstart >= offs) & (bstart < offs + padded)
    be = jnp.sum(jnp.where(sel, lane_e, 0), axis=1, keepdims=True)
    hit = jnp.sum(sel.astype(jnp.int32), axis=1, keepdims=True) > 0
    be_ref[...] = jnp.where(hit, be, -1).astype(jnp.int32)

    ci = lax.broadcasted_iota(jnp.int32, (TB, TB), 0)
    cj = lax.broadcasted_iota(jnp.int32, (TB, TB), 1)
    lstrict = (cj < ci).astype(jnp.float32)
    for c in range(nch):
        prior = jnp.dot(lstrict, ohs[c], preferred_element_type=jnp.float32)
        posf = jnp.sum(ohs[c] * (prior + coffs[c] + offs), axis=1, keepdims=True)
        pos_ref[pl.ds(c * TB, TB), :] = posf.astype(jnp.int32)


def _routing(idx_all):
    return pl.pallas_call(
        _routing_body,
        out_shape=[
            jax.ShapeDtypeStruct((A, 1), jnp.int32),
            jax.ShapeDtypeStruct((NB, 1), jnp.int32),
        ],
    )(idx_all)


def _dispatch_body(x_hbm, pos_hbm, xs_hbm, rowbuf, posbuf, sem):
    c = lax.axis_index("c")
    s = lax.axis_index("s")
    wid = s * 2 + c

    rows_per = A // NW
    chunk = 64
    for j in range(rows_per // chunk):
        a0 = wid * rows_per + j * chunk
        t0 = lax.rem(a0, T)
        pltpu.sync_copy(x_hbm.at[pl.ds(t0, chunk)], rowbuf)
        pltpu.sync_copy(pos_hbm.at[pl.ds(a0, chunk)], posbuf)
        pltpu.async_copy(rowbuf, xs_hbm.at[posbuf], sem).wait()


def _dispatch(xf, pos_flat):
    mesh = plsc.VectorSubcoreMesh(core_axis_name="c", subcore_axis_name="s")
    fn = pl.kernel(
        _dispatch_body,
        out_type=jax.ShapeDtypeStruct((BUF, H // 2), jnp.int32),
        mesh=mesh,
        scratch_types=[
            pltpu.VMEM((64, H // 2), jnp.int32),
            pltpu.VMEM((64,), jnp.int32),
            pltpu.SemaphoreType.DMA,
        ],
    )
    return fn(xf, pos_flat)


def _mlp_body(be_ref, xs_ref, w1_ref, b1_ref, w2_ref, b2_ref, ys_ref):
    b = pl.program_id(0)
    e = be_ref[b]

    @pl.when(e >= 0)
    def _():
        xb = xs_ref[...]
        h = jnp.dot(xb, w1_ref[0], preferred_element_type=jnp.float32)
        h = h + b1_ref[0]
        g = 0.5 * h * (1.0 + lax.erf(h * 0.7071067811865476))
        y = jnp.dot(g.astype(jnp.bfloat16), w2_ref[0],
                    preferred_element_type=jnp.float32)
        ys_ref[...] = y + b2_ref[0]


def _mlp(be, xs, W1, b1r, W2, b2r):
    def xin(b, be_ref):
        return (jnp.where(be_ref[b] >= 0, b, 0), 0)

    def ein(b, be_ref):
        return (jnp.maximum(be_ref[b], 0), 0, 0)

    def yout(b, be_ref):
        return (jnp.where(be_ref[b] >= 0, b, NB - 1), 0)

    grid_spec = pltpu.PrefetchScalarGridSpec(
        num_scalar_prefetch=1,
        grid=(NB,),
        in_specs=[
            pl.BlockSpec((BM, H), xin),
            pl.BlockSpec((1, H, I), ein),
            pl.BlockSpec((1, 1, I), ein),
            pl.BlockSpec((1, I, H), ein),
            pl.BlockSpec((1, 1, H), ein),
        ],
        out_specs=pl.BlockSpec((BM, H), yout),
    )
    return pl.pallas_call(
        _mlp_body,
        grid_spec=grid_spec,
        out_shape=jax.ShapeDtypeStruct((BUF, H), jnp.float32),
    )(be, xs, W1, b1r, W2, b2r)


def _combine_body(ys_hbm, pos_hbm, w0_hbm, w1_hbm, out_hbm,
                  i0buf, i1buf, w0buf, w1buf, bufa, bufb, sema, semb):
    c = lax.axis_index("c")
    s = lax.axis_index("s")
    wid = s * 2 + c

    rows_per = T // NW
    chunk = 32
    for j in range(rows_per // chunk):
        t0 = wid * rows_per + j * chunk
        pltpu.sync_copy(pos_hbm.at[pl.ds(t0, chunk)], i0buf)
        pltpu.sync_copy(pos_hbm.at[pl.ds(T + t0, chunk)], i1buf)
        pltpu.sync_copy(w0_hbm.at[pl.ds(t0, chunk)], w0buf)
        pltpu.sync_copy(w1_hbm.at[pl.ds(t0, chunk)], w1buf)
        cpa = pltpu.async_copy(ys_hbm.at[i0buf], bufa, sema)
        cpb = pltpu.async_copy(ys_hbm.at[i1buf], bufb, semb)
        cpa.wait()
        cpb.wait()

        def row(r, carry):
            wa = w0buf[r, :]
            wb = w1buf[r, :]
            for l in range(H // 16):
                sl = pl.ds(l * 16, 16)
                bufa[r, sl] = bufa[r, sl] * wa + bufb[r, sl] * wb
            return carry

        lax.fori_loop(0, chunk, row, 0)
        pltpu.sync_copy(bufa, out_hbm.at[pl.ds(t0, chunk)])


def _combine(ys, pos_flat, w0_flat, w1_flat):
    mesh = plsc.VectorSubcoreMesh(core_axis_name="c", subcore_axis_name="s")
    fn = pl.kernel(
        _combine_body,
        out_type=jax.ShapeDtypeStruct((T, H), jnp.float32),
        mesh=mesh,
        scratch_types=[
            pltpu.VMEM((32,), jnp.int32),
            pltpu.VMEM((32,), jnp.int32),
            pltpu.VMEM((32, 16), jnp.float32),
            pltpu.VMEM((32, 16), jnp.float32),
            pltpu.VMEM((32, H), jnp.float32),
            pltpu.VMEM((32, H), jnp.float32),
            pltpu.SemaphoreType.DMA,
            pltpu.SemaphoreType.DMA,
        ],
    )
    return fn(ys, pos_flat, w0_flat, w1_flat)


def kernel(x, w_gate, W1, b1, W2, b2):
    xf = x.reshape(T, H)
    i0, i1, w0, w1 = _gating(xf, w_gate)
    idx_all = jnp.concatenate([i0, i1], axis=0)
    pos, be = _routing(idx_all)
    pos_flat = pos.reshape(A)
    x_pack = lax.bitcast_convert_type(
        xf.astype(jnp.bfloat16).reshape(T, H // 2, 2), jnp.int32)
    xs_pack = _dispatch(x_pack, pos_flat)
    xs = lax.bitcast_convert_type(xs_pack, jnp.bfloat16).reshape(BUF, H)
    ys = _mlp(be.reshape(NB), xs,
              W1.astype(jnp.bfloat16), b1.reshape(E, 1, I),
              W2.astype(jnp.bfloat16), b2.reshape(E, 1, H))
    out = _combine(ys, pos_flat, w0, w1)
    return out.reshape(x.shape)


# speedup vs baseline: 2.1870x; 2.1870x over previous
import functools

import jax
import jax.numpy as jnp
from jax import lax
from jax.experimental import pallas as pl
from jax.experimental.pallas import tpu as pltpu
from jax.experimental.pallas import tpu_sc as plsc

T = 2048
H = 1024
E = 64
K = 2
I = 64
A = T * K
BM = 64
BUF = 8192
NB = BUF // BM
NW = 32
TB = 256
GB = 512


def _gating_body(x_ref, wg_ref, i0_ref, i1_ref, w0_ref, w1_ref):
    xb = x_ref[...]
    logits = jnp.dot(xb, wg_ref[...], preferred_element_type=jnp.float32)
    m = jnp.max(logits, axis=-1, keepdims=True)
    ex = jnp.exp(logits - m)
    raw = ex / jnp.sum(ex, axis=-1, keepdims=True)
    lane = lax.broadcasted_iota(jnp.int32, raw.shape, 1)
    p1 = jnp.max(raw, axis=-1, keepdims=True)
    a1 = jnp.min(jnp.where(raw == p1, lane, E), axis=-1, keepdims=True)
    masked = jnp.where(lane == a1, -1.0, raw)
    p2 = jnp.max(masked, axis=-1, keepdims=True)
    a2 = jnp.min(jnp.where(masked == p2, lane, E), axis=-1, keepdims=True)
    e2 = jnp.exp(p2 - p1)
    w0 = 1.0 / (1.0 + e2)
    i0_ref[...] = a1.astype(jnp.int32)
    i1_ref[...] = a2.astype(jnp.int32)
    w0_ref[...] = jnp.broadcast_to(w0, (GB, 16))
    w1_ref[...] = jnp.broadcast_to(1.0 - w0, (GB, 16))


def _gating(xf, w_gate):
    col_i = jax.ShapeDtypeStruct((T, 1), jnp.int32)
    spl_f = jax.ShapeDtypeStruct((T, 16), jnp.float32)
    return pl.pallas_call(
        _gating_body,
        grid=(T // GB,),
        in_specs=[
            pl.BlockSpec((GB, H), lambda i: (i, 0)),
            pl.BlockSpec((H, E), lambda i: (0, 0)),
        ],
        out_specs=[
            pl.BlockSpec((GB, 1), lambda i: (i, 0)),
            pl.BlockSpec((GB, 1), lambda i: (i, 0)),
            pl.BlockSpec((GB, 16), lambda i: (i, 0)),
            pl.BlockSpec((GB, 16), lambda i: (i, 0)),
        ],
        out_shape=[col_i, col_i, spl_f, spl_f],
    )(xf, w_gate)


def _routing_body(idx_ref, pos_ref, be_ref):
    lane = lax.broadcasted_iota(jnp.int32, (TB, E), 1)
    nch = A // TB
    ohs = []
    run = jnp.zeros((1, E), jnp.float32)
    coffs = []
    for c in range(nch):
        ohc = (idx_ref[pl.ds(c * TB, TB), :] == lane).astype(jnp.float32)
        ohs.append(ohc)
        coffs.append(run)
        run = run + jnp.sum(ohc, axis=0, keepdims=True)
    counts = run
    padded = jnp.ceil(counts / BM) * BM
    li = lax.broadcasted_iota(jnp.int32, (E, E), 0)
    lj = lax.broadcasted_iota(jnp.int32, (E, E), 1)
    ustrict = (li < lj).astype(jnp.float32)
    offs = jnp.dot(padded, ustrict, preferred_element_type=jnp.float32)

    bstart = (lax.broadcasted_iota(jnp.int32, (NB, E), 0) * BM).astype(jnp.float32)
    lane_e = lax.broadcasted_iota(jnp.int32, (NB, E), 1)
    sel = (bstart >= offs) & (bstart < offs + padded)
    be = jnp.sum(jnp.where(sel, lane_e, 0), axis=1, keepdims=True)
    hit = jnp.sum(sel.astype(jnp.int32), axis=1, keepdims=True) > 0
    be_ref[...] = jnp.where(hit, be, -1).astype(jnp.int32)

    ci = lax.broadcasted_iota(jnp.int32, (TB, TB), 0)
    cj = lax.broadcasted_iota(jnp.int32, (TB, TB), 1)
    lstrict = (cj < ci).astype(jnp.float32)
    for c in range(nch):
        prior = jnp.dot(lstrict, ohs[c], preferred_element_type=jnp.float32)
        posf = jnp.sum(ohs[c] * (prior + coffs[c] + offs), axis=1, keepdims=True)
        pos_ref[pl.ds(c * TB, TB), :] = posf.astype(jnp.int32)


def _routing(idx_all):
    return pl.pallas_call(
        _routing_body,
        out_shape=[
            jax.ShapeDtypeStruct((A, 1), jnp.int32),
            jax.ShapeDtypeStruct((NB, 1), jnp.int32),
        ],
    )(idx_all)


def _dispatch_body(x_hbm, pos_hbm, xs_hbm, rowbuf, posbuf, sem):
    c = lax.axis_index("c")
    s = lax.axis_index("s")
    wid = s * 2 + c

    rows_per = A // NW
    chunk = 64
    for j in range(rows_per // chunk):
        a0 = wid * rows_per + j * chunk
        t0 = lax.rem(a0, T)
        pltpu.sync_copy(x_hbm.at[pl.ds(t0, chunk)], rowbuf)
        pltpu.sync_copy(pos_hbm.at[pl.ds(a0, chunk)], posbuf)
        pltpu.async_copy(rowbuf, xs_hbm.at[posbuf], sem).wait()


def _dispatch(xf, pos_flat):
    mesh = plsc.VectorSubcoreMesh(core_axis_name="c", subcore_axis_name="s")
    fn = pl.kernel(
        _dispatch_body,
        out_type=jax.ShapeDtypeStruct((BUF, H // 2), jnp.int32),
        mesh=mesh,
        scratch_types=[
            pltpu.VMEM((64, H // 2), jnp.int32),
            pltpu.VMEM((64,), jnp.int32),
            pltpu.SemaphoreType.DMA,
        ],
    )
    return fn(xf, pos_flat)


P = 4


def _mlp_body(be_ref, xs_ref, *rest):
    w1_refs = rest[0:P]
    b1_refs = rest[P:2 * P]
    w2_refs = rest[2 * P:3 * P]
    b2_refs = rest[3 * P:4 * P]
    ys_ref = rest[4 * P]
    b = pl.program_id(0)
    xi = xs_ref[...]

    for p in range(P):
        e = be_ref[b * P + p]

        @pl.when(e >= 0)
        def _(p=p):
            xp = xi[p * BM:(p + 1) * BM, :]
            lo = lax.bitcast_convert_type(xp << 16, jnp.float32)
            hi = lax.bitcast_convert_type(xp & jnp.int32(-65536), jnp.float32)
            w1 = w1_refs[p][0]
            h = jnp.dot(lo.astype(jnp.bfloat16), w1[:H // 2, :],
                        preferred_element_type=jnp.float32)
            h = h + jnp.dot(hi.astype(jnp.bfloat16), w1[H // 2:, :],
                            preferred_element_type=jnp.float32)
            h = h + b1_refs[p][0]
            g = 0.5 * h * (1.0 + lax.erf(h * 0.7071067811865476))
            y = jnp.dot(g.astype(jnp.bfloat16), w2_refs[p][0],
                        preferred_element_type=jnp.float32)
            ys_ref[pl.ds(p * BM, BM), :] = y + b2_refs[p][0]


def _mlp(be, xs_pack, W1, b1r, W2, b2r):
    def ein(p):
        def f(b, be_ref):
            return (jnp.maximum(be_ref[b * P + p], 0), 0, 0)
        return f

    grid_spec = pltpu.PrefetchScalarGridSpec(
        num_scalar_prefetch=1,
        grid=(NB // P,),
        in_specs=[
            pl.BlockSpec((P * BM, H // 2), lambda b, be_ref: (b, 0)),
            *[pl.BlockSpec((1, H, I), ein(p)) for p in range(P)],
            *[pl.BlockSpec((1, 1, I), ein(p)) for p in range(P)],
            *[pl.BlockSpec((1, I, H), ein(p)) for p in range(P)],
            *[pl.BlockSpec((1, 1, H), ein(p)) for p in range(P)],
        ],
        out_specs=pl.BlockSpec((P * BM, H), lambda b, be_ref: (b, 0)),
    )
    return pl.pallas_call(
        _mlp_body,
        grid_spec=grid_spec,
        out_shape=jax.ShapeDtypeStruct((BUF, H), jnp.float32),
    )(be, xs_pack,
      *([W1] * P), *([b1r] * P), *([W2] * P), *([b2r] * P))


def _combine_body(ys_hbm, pos_hbm, w0_hbm, w1_hbm, out_hbm,
                  i0buf, i1buf, w0buf, w1buf, bufa, bufb, sema, semb):
    c = lax.axis_index("c")
    s = lax.axis_index("s")
    wid = s * 2 + c

    rows_per = T // NW
    chunk = 32
    for j in range(rows_per // chunk):
        t0 = wid * rows_per + j * chunk
        pltpu.sync_copy(pos_hbm.at[pl.ds(t0, chunk)], i0buf)
        pltpu.sync_copy(pos_hbm.at[pl.ds(T + t0, chunk)], i1buf)
        pltpu.sync_copy(w0_hbm.at[pl.ds(t0, chunk)], w0buf)
        pltpu.sync_copy(w1_hbm.at[pl.ds(t0, chunk)], w1buf)
        cpa = pltpu.async_copy(ys_hbm.at[i0buf], bufa, sema)
        cpb = pltpu.async_copy(ys_hbm.at[i1buf], bufb, semb)
        cpa.wait()
        cpb.wait()

        def row(r, carry):
            wa = w0buf[r, :]
            wb = w1buf[r, :]
            for l in range(H // 16):
                sl = pl.ds(l * 16, 16)
                bufa[r, sl] = bufa[r, sl] * wa + bufb[r, sl] * wb
            return carry

        lax.fori_loop(0, chunk, row, 0)
        pltpu.sync_copy(bufa, out_hbm.at[pl.ds(t0, chunk)])


def _combine(ys, pos_flat, w0_flat, w1_flat):
    mesh = plsc.VectorSubcoreMesh(core_axis_name="c", subcore_axis_name="s")
    fn = pl.kernel(
        _combine_body,
        out_type=jax.ShapeDtypeStruct((T, H), jnp.float32),
        mesh=mesh,
        scratch_types=[
            pltpu.VMEM((32,), jnp.int32),
            pltpu.VMEM((32,), jnp.int32),
            pltpu.VMEM((32, 16), jnp.float32),
            pltpu.VMEM((32, 16), jnp.float32),
            pltpu.VMEM((32, H), jnp.float32),
            pltpu.VMEM((32, H), jnp.float32),
            pltpu.SemaphoreType.DMA,
            pltpu.SemaphoreType.DMA,
        ],
    )
    return fn(ys, pos_flat, w0_flat, w1_flat)


def kernel(x, w_gate, W1, b1, W2, b2):
    xf = x.reshape(T, H)
    i0, i1, w0, w1 = _gating(xf, w_gate)
    idx_all = jnp.concatenate([i0, i1], axis=0)
    pos, be = _routing(idx_all)
    pos_flat = pos.reshape(A)
    lo = lax.bitcast_convert_type(
        xf[:, :H // 2].astype(jnp.bfloat16), jnp.uint16).astype(jnp.int32)
    hi = lax.bitcast_convert_type(
        xf[:, H // 2:].astype(jnp.bfloat16), jnp.uint16).astype(jnp.int32)
    x_pack = lo | (hi << 16)
    xs_pack = _dispatch(x_pack, pos_flat)
    ys = _mlp(be.reshape(NB), xs_pack,
              W1.astype(jnp.bfloat16), b1.reshape(E, 1, I),
              W2.astype(jnp.bfloat16), b2.reshape(E, 1, H))
    out = _combine(ys, pos_flat, w0, w1)
    return out.reshape(x.shape)


# speedup vs baseline: 2.3286x; 1.0647x over previous
import jax
import jax.numpy as jnp
from jax import lax
from jax.experimental import pallas as pl
from jax.experimental.pallas import tpu as pltpu
from jax.experimental.pallas import tpu_sc as plsc

T = 2048
H = 1024
E = 64
K = 2
I = 64
A = T * K
BM = 64
BUF = 8192
NB = BUF // BM
P = 4
NW = 32
TB = 256
GB = 512


def _gate_route_body(x_ref, wg_ref, wall_ref, pos_ref, be_ref, idx_sc):
    step = pl.program_id(0)

    @pl.when(step < T // GB)
    def _gate():
        xb = x_ref[...]
        logits = jnp.dot(xb, wg_ref[...], preferred_element_type=jnp.float32)
        m = jnp.max(logits, axis=-1, keepdims=True)
        ex = jnp.exp(logits - m)
        raw = ex / jnp.sum(ex, axis=-1, keepdims=True)
        lane = lax.broadcasted_iota(jnp.int32, raw.shape, 1)
        p1 = jnp.max(raw, axis=-1, keepdims=True)
        a1 = jnp.min(jnp.where(raw == p1, lane, E), axis=-1, keepdims=True)
        masked = jnp.where(lane == a1, -1.0, raw)
        p2 = jnp.max(masked, axis=-1, keepdims=True)
        a2 = jnp.min(jnp.where(masked == p2, lane, E), axis=-1, keepdims=True)
        e2 = jnp.exp(p2 - p1)
        w0 = 1.0 / (1.0 + e2)
        r0 = step * GB
        idx_sc[pl.ds(r0, GB), :] = a1.astype(jnp.int32)
        idx_sc[pl.ds(T + r0, GB), :] = a2.astype(jnp.int32)
        wall_ref[pl.ds(r0, GB), :] = jnp.broadcast_to(w0, (GB, 128))
        wall_ref[pl.ds(T + r0, GB), :] = jnp.broadcast_to(1.0 - w0, (GB, 128))

    @pl.when(step == T // GB)
    def _route():
        lane = lax.broadcasted_iota(jnp.int32, (TB, E), 1)
        nch = A // TB
        ohs = []
        run = jnp.zeros((1, E), jnp.float32)
        coffs = []
        for c in range(nch):
            ohc = (idx_sc[pl.ds(c * TB, TB), :] == lane).astype(jnp.float32)
            ohs.append(ohc)
            coffs.append(run)
            run = run + jnp.sum(ohc, axis=0, keepdims=True)
        counts = run
        padded = jnp.ceil(counts / BM) * BM
        li = lax.broadcasted_iota(jnp.int32, (E, E), 0)
        lj = lax.broadcasted_iota(jnp.int32, (E, E), 1)
        ustrict = (li < lj).astype(jnp.float32)
        offs = jnp.dot(padded, ustrict, preferred_element_type=jnp.float32)

        bstart = (lax.broadcasted_iota(jnp.int32, (NB, E), 0) * BM).astype(
            jnp.float32)
        lane_e = lax.broadcasted_iota(jnp.int32, (NB, E), 1)
        sel = (bstart >= offs) & (bstart < offs + padded)
        be = jnp.sum(jnp.where(sel, lane_e, 0), axis=1, keepdims=True)
        hit = jnp.sum(sel.astype(jnp.int32), axis=1, keepdims=True) > 0
        be_ref[...] = jnp.where(hit, be, -1).astype(jnp.int32)

        ci = lax.broadcasted_iota(jnp.int32, (TB, TB), 0)
        cj = lax.broadcasted_iota(jnp.int32, (TB, TB), 1)
        lstrict = (cj < ci).astype(jnp.float32)
        for c in range(nch):
            prior = jnp.dot(lstrict, ohs[c], preferred_element_type=jnp.float32)
            posf = jnp.sum(ohs[c] * (prior + coffs[c] + offs), axis=1,
                           keepdims=True)
            pos_ref[pl.ds(c * TB, TB), :] = posf.astype(jnp.int32)


def _gate_route(xf, w_gate):
    nsteps = T // GB + 1
    return pl.pallas_call(
        _gate_route_body,
        grid=(nsteps,),
        in_specs=[
            pl.BlockSpec((GB, H), lambda i: (jnp.minimum(i, T // GB - 1), 0)),
            pl.BlockSpec((H, E), lambda i: (0, 0)),
        ],
        out_specs=[
            pl.BlockSpec((A, 128), lambda i: (0, 0)),
            pl.BlockSpec((A, 1), lambda i: (0, 0)),
            pl.BlockSpec((NB, 1), lambda i: (0, 0)),
        ],
        out_shape=[
            jax.ShapeDtypeStruct((A, 128), jnp.float32),
            jax.ShapeDtypeStruct((A, 1), jnp.int32),
            jax.ShapeDtypeStruct((NB, 1), jnp.int32),
        ],
        scratch_shapes=[pltpu.VMEM((A, 1), jnp.int32)],
    )(xf, w_gate)


def _dispatch_body(x_hbm, pos_hbm, w_hbm, xs_hbm, ws_hbm,
                   rowbuf, posbuf, wbuf, sem, sem2):
    c = lax.axis_index("c")
    s = lax.axis_index("s")
    wid = s * 2 + c

    rows_per = A // NW
    chunk = 64
    for j in range(rows_per // chunk):
        a0 = wid * rows_per + j * chunk
        t0 = lax.rem(a0, T)
        pltpu.sync_copy(x_hbm.at[pl.ds(t0, chunk)], rowbuf)
        pltpu.sync_copy(pos_hbm.at[pl.ds(a0, chunk)], posbuf)
        pltpu.sync_copy(w_hbm.at[pl.ds(a0, chunk)], wbuf)
        cp1 = pltpu.async_copy(rowbuf, xs_hbm.at[posbuf], sem)
        cp2 = pltpu.async_copy(wbuf, ws_hbm.at[posbuf], sem2)
        cp1.wait()
        cp2.wait()


def _dispatch(x_pack, pos_flat, w_all):
    mesh = plsc.VectorSubcoreMesh(core_axis_name="c", subcore_axis_name="s")
    fn = pl.kernel(
        _dispatch_body,
        out_type=(
            jax.ShapeDtypeStruct((BUF, H // 2), jnp.int32),
            jax.ShapeDtypeStruct((BUF, 128), jnp.float32),
        ),
        mesh=mesh,
        scratch_types=[
            pltpu.VMEM((64, H // 2), jnp.int32),
            pltpu.VMEM((64,), jnp.int32),
            pltpu.VMEM((64, 128), jnp.float32),
            pltpu.SemaphoreType.DMA,
            pltpu.SemaphoreType.DMA,
        ],
    )
    return fn(x_pack, pos_flat, w_all)


def _mlp_body(be_ref, xs_ref, ws_ref, *rest):
    w1_refs = rest[0:P]
    b1_refs = rest[P:2 * P]
    w2_refs = rest[2 * P:3 * P]
    b2_refs = rest[3 * P:4 * P]
    ys_ref = rest[4 * P]
    xi = xs_ref[...]
    wsv = ws_ref[...]

    for p in range(P):
        xp = xi[p * BM:(p + 1) * BM, :]
        lo = lax.bitcast_convert_type(xp << 16, jnp.float32)
        hi = lax.bitcast_convert_type(xp & jnp.int32(-65536), jnp.float32)
        w1 = w1_refs[p][0]
        h = jnp.dot(lo.astype(jnp.bfloat16), w1[:H // 2, :],
                    preferred_element_type=jnp.float32)
        h = h + jnp.dot(hi.astype(jnp.bfloat16), w1[H // 2:, :],
                        preferred_element_type=jnp.float32)
        h = h + b1_refs[p][0]
        g = 0.5 * h * (1.0 + lax.erf(h * 0.7071067811865476))
        y = jnp.dot(g.astype(jnp.bfloat16), w2_refs[p][0],
                    preferred_element_type=jnp.float32)
        y = (y + b2_refs[p][0]) * wsv[p * BM:(p + 1) * BM, 0:1]
        ys_ref[pl.ds(p * BM, BM), :] = y


def _mlp(be, xs_pack, ws, W1, b1r, W2, b2r):
    def ein(p):
        def f(b, be_ref):
            return (jnp.maximum(be_ref[b * P + p], 0), 0, 0)
        return f

    grid_spec = pltpu.PrefetchScalarGridSpec(
        num_scalar_prefetch=1,
        grid=(NB // P,),
        in_specs=[
            pl.BlockSpec((P * BM, H // 2), lambda b, be_ref: (b, 0)),
            pl.BlockSpec((P * BM, 128), lambda b, be_ref: (b, 0)),
            *[pl.BlockSpec((1, H, I), ein(p)) for p in range(P)],
            *[pl.BlockSpec((1, 1, I), ein(p)) for p in range(P)],
            *[pl.BlockSpec((1, I, H), ein(p)) for p in range(P)],
            *[pl.BlockSpec((1, 1, H), ein(p)) for p in range(P)],
        ],
        out_specs=pl.BlockSpec((P * BM, H), lambda b, be_ref: (b, 0)),
    )
    return pl.pallas_call(
        _mlp_body,
        grid_spec=grid_spec,
        out_shape=jax.ShapeDtypeStruct((BUF, H), jnp.float32),
    )(be, xs_pack, ws,
      *([W1] * P), *([b1r] * P), *([W2] * P), *([b2r] * P))


def _combine_body(ys_hbm, pos_hbm, out_hbm, i0buf, i1buf, bufa, bufb,
                  sema, semb):
    c = lax.axis_index("c")
    s = lax.axis_index("s")
    wid = s * 2 + c

    rows_per = T // NW
    chunk = 32
    for j in range(rows_per // chunk):
        t0 = wid * rows_per + j * chunk
        pltpu.sync_copy(pos_hbm.at[pl.ds(t0, chunk)], i0buf)
        pltpu.sync_copy(pos_hbm.at[pl.ds(T + t0, chunk)], i1buf)
        cpa = pltpu.async_copy(ys_hbm.at[i0buf], bufa, sema)
        cpb = pltpu.async_copy(ys_hbm.at[i1buf], bufb, semb)
        cpa.wait()
        cpb.wait()

        def row(r, carry):
            for l in range(H // 16):
                sl = pl.ds(l * 16, 16)
                bufa[r, sl] = bufa[r, sl] + bufb[r, sl]
            return carry

        lax.fori_loop(0, chunk, row, 0)
        pltpu.sync_copy(bufa, out_hbm.at[pl.ds(t0, chunk)])


def _combine(ys, pos_flat):
    mesh = plsc.VectorSubcoreMesh(core_axis_name="c", subcore_axis_name="s")
    fn = pl.kernel(
        _combine_body,
        out_type=jax.ShapeDtypeStruct((T, H), jnp.float32),
        mesh=mesh,
        scratch_types=[
            pltpu.VMEM((32,), jnp.int32),
            pltpu.VMEM((32,), jnp.int32),
            pltpu.VMEM((32, H), jnp.float32),
            pltpu.VMEM((32, H), jnp.float32),
            pltpu.SemaphoreType.DMA,
            pltpu.SemaphoreType.DMA,
        ],
    )
    return fn(ys, pos_flat)


def kernel(x, w_gate, W1, b1, W2, b2):
    xf = x.reshape(T, H)
    w_all, pos, be = _gate_route(xf, w_gate)
    pos_flat = pos.reshape(A)
    lo = lax.bitcast_convert_type(
        xf[:, :H // 2].astype(jnp.bfloat16), jnp.uint16).astype(jnp.int32)
    hi = lax.bitcast_convert_type(
        xf[:, H // 2:].astype(jnp.bfloat16), jnp.uint16).astype(jnp.int32)
    x_pack = lo | (hi << 16)
    xs_pack, ws = _dispatch(x_pack, pos_flat, w_all)
    ys = _mlp(be.reshape(NB), xs_pack, ws,
              W1.astype(jnp.bfloat16), b1.reshape(E, 1, I),
              W2.astype(jnp.bfloat16), b2.reshape(E, 1, H))
    out = _combine(ys, pos_flat)
    return out.reshape(x.shape)


# speedup vs baseline: 2.4539x; 1.0538x over previous
import jax
import jax.numpy as jnp
from jax import lax
from jax.experimental import pallas as pl
from jax.experimental.pallas import tpu as pltpu
from jax.experimental.pallas import tpu_sc as plsc

T = 2048
H = 1024
E = 64
K = 2
I = 64
A = T * K
BM = 64
BUF = 8192
NB = BUF // BM
P = 4
NW = 32
TB = 256
GB = 512


def _gate_route_body(x_ref, wg_ref, wall_ref, pos_ref, be_ref, idx_sc):
    step = pl.program_id(0)

    @pl.when(step < T // GB)
    def _gate():
        xb = x_ref[...]
        logits = jnp.dot(xb, wg_ref[...], preferred_element_type=jnp.float32)
        m = jnp.max(logits, axis=-1, keepdims=True)
        ex = jnp.exp(logits - m)
        raw = ex / jnp.sum(ex, axis=-1, keepdims=True)
        lane = lax.broadcasted_iota(jnp.int32, raw.shape, 1)
        p1 = jnp.max(raw, axis=-1, keepdims=True)
        a1 = jnp.min(jnp.where(raw == p1, lane, E), axis=-1, keepdims=True)
        masked = jnp.where(lane == a1, -1.0, raw)
        p2 = jnp.max(masked, axis=-1, keepdims=True)
        a2 = jnp.min(jnp.where(masked == p2, lane, E), axis=-1, keepdims=True)
        e2 = jnp.exp(p2 - p1)
        w0 = 1.0 / (1.0 + e2)
        r0 = step * GB
        idx_sc[pl.ds(r0, GB), :] = a1.astype(jnp.int32)
        idx_sc[pl.ds(T + r0, GB), :] = a2.astype(jnp.int32)
        wall_ref[pl.ds(r0, GB), :] = jnp.broadcast_to(w0, (GB, 128))
        wall_ref[pl.ds(T + r0, GB), :] = jnp.broadcast_to(1.0 - w0, (GB, 128))

    @pl.when(step == T // GB)
    def _route():
        lane = lax.broadcasted_iota(jnp.int32, (TB, E), 1)
        nch = A // TB
        ohs = []
        run = jnp.zeros((1, E), jnp.float32)
        coffs = []
        for c in range(nch):
            ohc = (idx_sc[pl.ds(c * TB, TB), :] == lane).astype(jnp.float32)
            ohs.append(ohc)
            coffs.append(run)
            run = run + jnp.sum(ohc, axis=0, keepdims=True)
        counts = run
        padded = jnp.ceil(counts / BM) * BM
        li = lax.broadcasted_iota(jnp.int32, (E, E), 0)
        lj = lax.broadcasted_iota(jnp.int32, (E, E), 1)
        ustrict = (li < lj).astype(jnp.float32)
        offs = jnp.dot(padded, ustrict, preferred_element_type=jnp.float32)

        bstart = (lax.broadcasted_iota(jnp.int32, (NB, E), 0) * BM).astype(
            jnp.float32)
        lane_e = lax.broadcasted_iota(jnp.int32, (NB, E), 1)
        sel = (bstart >= offs) & (bstart < offs + padded)
        be = jnp.sum(jnp.where(sel, lane_e, 0), axis=1, keepdims=True)
        hit = jnp.sum(sel.astype(jnp.int32), axis=1, keepdims=True) > 0
        be_ref[...] = jnp.where(hit, be, -1).astype(jnp.int32)

        ci = lax.broadcasted_iota(jnp.int32, (TB, TB), 0)
        cj = lax.broadcasted_iota(jnp.int32, (TB, TB), 1)
        lstrict = (cj < ci).astype(jnp.float32)
        for c in range(nch):
            prior = jnp.dot(lstrict, ohs[c], preferred_element_type=jnp.float32)
            posf = jnp.sum(ohs[c] * (prior + coffs[c] + offs), axis=1,
                           keepdims=True)
            pos_ref[pl.ds(c * TB, TB), :] = posf.astype(jnp.int32)


def _gate_route(xf, w_gate):
    nsteps = T // GB + 1
    return pl.pallas_call(
        _gate_route_body,
        grid=(nsteps,),
        in_specs=[
            pl.BlockSpec((GB, H), lambda i: (jnp.minimum(i, T // GB - 1), 0)),
            pl.BlockSpec((H, E), lambda i: (0, 0)),
        ],
        out_specs=[
            pl.BlockSpec((A, 128), lambda i: (0, 0)),
            pl.BlockSpec((A, 1), lambda i: (0, 0)),
            pl.BlockSpec((NB, 1), lambda i: (0, 0)),
        ],
        out_shape=[
            jax.ShapeDtypeStruct((A, 128), jnp.float32),
            jax.ShapeDtypeStruct((A, 1), jnp.int32),
            jax.ShapeDtypeStruct((NB, 1), jnp.int32),
        ],
        scratch_shapes=[pltpu.VMEM((A, 1), jnp.int32)],
    )(xf, w_gate)


def _dispatch_body(x_hbm, pos_hbm, w_hbm, xs_hbm, ws_hbm,
                   rowbuf, posbuf, wbuf, sem, sem2):
    c = lax.axis_index("c")
    s = lax.axis_index("s")
    wid = s * 2 + c

    rows_per = A // NW
    chunk = 64
    for j in range(rows_per // chunk):
        a0 = wid * rows_per + j * chunk
        t0 = lax.rem(a0, T)
        pltpu.sync_copy(x_hbm.at[pl.ds(t0, chunk)], rowbuf)
        pltpu.sync_copy(pos_hbm.at[pl.ds(a0, chunk)], posbuf)
        pltpu.sync_copy(w_hbm.at[pl.ds(a0, chunk)], wbuf)
        cp1 = pltpu.async_copy(rowbuf, xs_hbm.at[posbuf], sem)
        cp2 = pltpu.async_copy(wbuf, ws_hbm.at[posbuf], sem2)
        cp1.wait()
        cp2.wait()


def _dispatch(x_pack, pos_flat, w_all):
    mesh = plsc.VectorSubcoreMesh(core_axis_name="c", subcore_axis_name="s")
    fn = pl.kernel(
        _dispatch_body,
        out_type=(
            jax.ShapeDtypeStruct((BUF, H // 2), jnp.int32),
            jax.ShapeDtypeStruct((BUF, 128), jnp.float32),
        ),
        mesh=mesh,
        scratch_types=[
            pltpu.VMEM((64, H // 2), jnp.int32),
            pltpu.VMEM((64,), jnp.int32),
            pltpu.VMEM((64, 128), jnp.float32),
            pltpu.SemaphoreType.DMA,
            pltpu.SemaphoreType.DMA,
        ],
    )
    return fn(x_pack, pos_flat, w_all)


def _mlp_body(be_ref, xs_ref, ws_ref, *rest):
    w1_refs = rest[0:P]
    b1_refs = rest[P:2 * P]
    w2_refs = rest[2 * P:3 * P]
    b2_refs = rest[3 * P:4 * P]
    ys_ref = rest[4 * P]
    xi = xs_ref[...]
    wsv = ws_ref[...]

    hs = []
    for p in range(P):
        xp = xi[p * BM:(p + 1) * BM, :]
        lo = lax.bitcast_convert_type(xp << 16, jnp.float32)
        hi = lax.bitcast_convert_type(xp & jnp.int32(-65536), jnp.float32)
        w1 = w1_refs[p][0]
        h = jnp.dot(lo.astype(jnp.bfloat16), w1[:H // 2, :],
                    preferred_element_type=jnp.float32)
        h = h + jnp.dot(hi.astype(jnp.bfloat16), w1[H // 2:, :],
                        preferred_element_type=jnp.float32)
        hs.append(h + b1_refs[p][0])
    gs = [0.5 * h * (1.0 + lax.erf(h * 0.7071067811865476)) for h in hs]
    for p in range(P):
        y = jnp.dot(gs[p].astype(jnp.bfloat16), w2_refs[p][0],
                    preferred_element_type=jnp.float32)
        y = (y + b2_refs[p][0]) * wsv[p * BM:(p + 1) * BM, 0:1]
        ys_ref[pl.ds(p * BM, BM), :] = y


def _mlp(be, xs_pack, ws, W1, b1r, W2, b2r):
    def ein(p):
        def f(b, be_ref):
            return (jnp.maximum(be_ref[b * P + p], 0), 0, 0)
        return f

    grid_spec = pltpu.PrefetchScalarGridSpec(
        num_scalar_prefetch=1,
        grid=(NB // P,),
        in_specs=[
            pl.BlockSpec((P * BM, H // 2), lambda b, be_ref: (b, 0)),
            pl.BlockSpec((P * BM, 128), lambda b, be_ref: (b, 0)),
            *[pl.BlockSpec((1, H, I), ein(p)) for p in range(P)],
            *[pl.BlockSpec((1, 1, I), ein(p)) for p in range(P)],
            *[pl.BlockSpec((1, I, H), ein(p)) for p in range(P)],
            *[pl.BlockSpec((1, 1, H), ein(p)) for p in range(P)],
        ],
        out_specs=pl.BlockSpec((P * BM, H), lambda b, be_ref: (b, 0)),
    )
    return pl.pallas_call(
        _mlp_body,
        grid_spec=grid_spec,
        out_shape=jax.ShapeDtypeStruct((BUF, H), jnp.float32),
    )(be, xs_pack, ws,
      *([W1] * P), *([b1r] * P), *([W2] * P), *([b2r] * P))


def _combine_body(ys_hbm, pos_hbm, out_hbm, i0buf, i1buf, bufa, bufb,
                  sema, semb):
    c = lax.axis_index("c")
    s = lax.axis_index("s")
    wid = s * 2 + c

    rows_per = T // NW
    chunk = 32

    for j in range(rows_per // chunk):
        t0 = wid * rows_per + j * chunk
        pltpu.sync_copy(pos_hbm.at[pl.ds(t0, chunk)], i0buf)
        pltpu.sync_copy(pos_hbm.at[pl.ds(T + t0, chunk)], i1buf)
        cpa = pltpu.async_copy(ys_hbm.at[i0buf], bufa, sema)
        cpb = pltpu.async_copy(ys_hbm.at[i1buf], bufb, semb)
        cpa.wait()
        cpb.wait()

        def row(r, carry):
            for l in range(H // 16):
                sl = pl.ds(l * 16, 16)
                bufa[r, sl] = bufa[r, sl] + bufb[r, sl]
            return carry

        lax.fori_loop(0, chunk, row, 0)
        pltpu.sync_copy(bufa, out_hbm.at[pl.ds(t0, chunk)])


def _combine(ys, pos_flat):
    mesh = plsc.VectorSubcoreMesh(core_axis_name="c", subcore_axis_name="s")
    fn = pl.kernel(
        _combine_body,
        out_type=jax.ShapeDtypeStruct((T, H), jnp.float32),
        mesh=mesh,
        scratch_types=[
            pltpu.VMEM((32,), jnp.int32),
            pltpu.VMEM((32,), jnp.int32),
            pltpu.VMEM((32, H), jnp.float32),
            pltpu.VMEM((32, H), jnp.float32),
            pltpu.SemaphoreType.DMA,
            pltpu.SemaphoreType.DMA,
        ],
    )
    return fn(ys, pos_flat)


def kernel(x, w_gate, W1, b1, W2, b2):
    xf = x.reshape(T, H)
    w_all, pos, be = _gate_route(xf, w_gate)
    pos_flat = pos.reshape(A)
    lo = lax.bitcast_convert_type(
        xf[:, :H // 2].astype(jnp.bfloat16), jnp.uint16).astype(jnp.int32)
    hi = lax.bitcast_convert_type(
        xf[:, H // 2:].astype(jnp.bfloat16), jnp.uint16).astype(jnp.int32)
    x_pack = lo | (hi << 16)
    xs_pack, ws = _dispatch(x_pack, pos_flat, w_all)
    ys = _mlp(be.reshape(NB), xs_pack, ws,
              W1.astype(jnp.bfloat16), b1.reshape(E, 1, I),
              W2.astype(jnp.bfloat16), b2.reshape(E, 1, H))
    out = _combine(ys, pos_flat)
    return out.reshape(x.shape)


# speedup vs baseline: 2.5937x; 1.0570x over previous
import jax
import jax.numpy as jnp
from jax import lax
from jax.experimental import pallas as pl
from jax.experimental.pallas import tpu as pltpu
from jax.experimental.pallas import tpu_sc as plsc

T = 2048
H = 1024
E = 64
K = 2
I = 64
A = T * K
BM = 64
BUF = 8192
NB = BUF // BM
P = 4
NW = 32
TB = 256
GB = 512


def _gate_route_body(x_ref, wg_ref, wall_ref, pos_ref, be_ref, idx_sc):
    step = pl.program_id(0)

    @pl.when(step < T // GB)
    def _gate():
        xb = x_ref[...]
        logits = jnp.dot(xb, wg_ref[...], preferred_element_type=jnp.float32)
        m = jnp.max(logits, axis=-1, keepdims=True)
        ex = jnp.exp(logits - m)
        raw = ex / jnp.sum(ex, axis=-1, keepdims=True)
        lane = lax.broadcasted_iota(jnp.int32, raw.shape, 1)
        p1 = jnp.max(raw, axis=-1, keepdims=True)
        a1 = jnp.min(jnp.where(raw == p1, lane, E), axis=-1, keepdims=True)
        masked = jnp.where(lane == a1, -1.0, raw)
        p2 = jnp.max(masked, axis=-1, keepdims=True)
        a2 = jnp.min(jnp.where(masked == p2, lane, E), axis=-1, keepdims=True)
        e2 = jnp.exp(p2 - p1)
        w0 = 1.0 / (1.0 + e2)
        r0 = step * GB
        idx_sc[pl.ds(r0, GB), :] = a1.astype(jnp.int32)
        idx_sc[pl.ds(T + r0, GB), :] = a2.astype(jnp.int32)
        wall_ref[pl.ds(r0, GB), :] = w0
        wall_ref[pl.ds(T + r0, GB), :] = 1.0 - w0

    @pl.when(step == T // GB)
    def _route():
        lane = lax.broadcasted_iota(jnp.int32, (TB, E), 1)
        nch = A // TB
        ohs = []
        run = jnp.zeros((1, E), jnp.float32)
        coffs = []
        for c in range(nch):
            ohc = (idx_sc[pl.ds(c * TB, TB), :] == lane).astype(jnp.float32)
            ohs.append(ohc)
            coffs.append(run)
            run = run + jnp.sum(ohc, axis=0, keepdims=True)
        counts = run
        padded = jnp.ceil(counts / BM) * BM
        li = lax.broadcasted_iota(jnp.int32, (E, E), 0)
        lj = lax.broadcasted_iota(jnp.int32, (E, E), 1)
        ustrict = (li < lj).astype(jnp.float32)
        offs = jnp.dot(padded, ustrict, preferred_element_type=jnp.float32)

        bstart = (lax.broadcasted_iota(jnp.int32, (NB, E), 0) * BM).astype(
            jnp.float32)
        lane_e = lax.broadcasted_iota(jnp.int32, (NB, E), 1)
        sel = (bstart >= offs) & (bstart < offs + padded)
        be = jnp.sum(jnp.where(sel, lane_e, 0), axis=1, keepdims=True)
        hit = jnp.sum(sel.astype(jnp.int32), axis=1, keepdims=True) > 0
        be_ref[...] = jnp.where(hit, be, -1).astype(jnp.int32)

        ci = lax.broadcasted_iota(jnp.int32, (TB, TB), 0)
        cj = lax.broadcasted_iota(jnp.int32, (TB, TB), 1)
        lstrict = (cj < ci).astype(jnp.float32)
        for c in range(nch):
            prior = jnp.dot(lstrict, ohs[c], preferred_element_type=jnp.float32)
            posf = jnp.sum(ohs[c] * (prior + coffs[c] + offs), axis=1,
                           keepdims=True)
            pos_ref[pl.ds(c * TB, TB), :] = posf.astype(jnp.int32)


def _gate_route(xf, w_gate):
    nsteps = T // GB + 1
    return pl.pallas_call(
        _gate_route_body,
        grid=(nsteps,),
        in_specs=[
            pl.BlockSpec((GB, H), lambda i: (jnp.minimum(i, T // GB - 1), 0)),
            pl.BlockSpec((H, E), lambda i: (0, 0)),
        ],
        out_specs=[
            pl.BlockSpec((A, 1), lambda i: (0, 0)),
            pl.BlockSpec((A, 1), lambda i: (0, 0)),
            pl.BlockSpec((NB, 1), lambda i: (0, 0)),
        ],
        out_shape=[
            jax.ShapeDtypeStruct((A, 1), jnp.float32),
            jax.ShapeDtypeStruct((A, 1), jnp.int32),
            jax.ShapeDtypeStruct((NB, 1), jnp.int32),
        ],
        scratch_shapes=[pltpu.VMEM((A, 1), jnp.int32)],
    )(xf, w_gate)


def _dispatch_body(x_hbm, pos_hbm, xs_hbm, rowbuf, posbuf, sem):
    c = lax.axis_index("c")
    s = lax.axis_index("s")
    wid = s * 2 + c

    rows_per = A // NW
    chunk = 64
    for j in range(rows_per // chunk):
        a0 = wid * rows_per + j * chunk
        t0 = lax.rem(a0, T)
        pltpu.sync_copy(x_hbm.at[pl.ds(t0, chunk)], rowbuf)
        pltpu.sync_copy(pos_hbm.at[pl.ds(a0, chunk)], posbuf)
        pltpu.async_copy(rowbuf, xs_hbm.at[posbuf], sem).wait()


def _dispatch(x_pack, pos_flat):
    mesh = plsc.VectorSubcoreMesh(core_axis_name="c", subcore_axis_name="s")
    fn = pl.kernel(
        _dispatch_body,
        out_type=jax.ShapeDtypeStruct((BUF, H // 2), jnp.int32),
        mesh=mesh,
        scratch_types=[
            pltpu.VMEM((64, H // 2), jnp.int32),
            pltpu.VMEM((64,), jnp.int32),
            pltpu.SemaphoreType.DMA,
        ],
    )
    return fn(x_pack, pos_flat)


def _mlp_body(be_ref, xs_ref, *rest):
    w1_refs = rest[0:P]
    b1_refs = rest[P:2 * P]
    w2_refs = rest[2 * P:3 * P]
    b2_refs = rest[3 * P:4 * P]
    ys_ref = rest[4 * P]
    xi = xs_ref[...]

    hs = []
    for p in range(P):
        xp = xi[p * BM:(p + 1) * BM, :]
        lo = lax.bitcast_convert_type(xp << 16, jnp.float32)
        hi = lax.bitcast_convert_type(xp & jnp.int32(-65536), jnp.float32)
        w1 = w1_refs[p][0]
        h = jnp.dot(lo.astype(jnp.bfloat16), w1[:H // 2, :],
                    preferred_element_type=jnp.float32)
        h = h + jnp.dot(hi.astype(jnp.bfloat16), w1[H // 2:, :],
                        preferred_element_type=jnp.float32)
        hs.append(h + b1_refs[p][0])
    gs = [0.5 * h * (1.0 + lax.erf(h * 0.7071067811865476)) for h in hs]
    for p in range(P):
        y = jnp.dot(gs[p].astype(jnp.bfloat16), w2_refs[p][0],
                    preferred_element_type=jnp.float32)
        y = y + b2_refs[p][0]
        ylo = lax.bitcast_convert_type(
            y[:, :H // 2].astype(jnp.bfloat16), jnp.uint16).astype(jnp.int32)
        yhi = lax.bitcast_convert_type(
            y[:, H // 2:].astype(jnp.bfloat16), jnp.uint16).astype(jnp.int32)
        ys_ref[pl.ds(p * BM, BM), :] = ylo | (yhi << 16)


def _mlp(be, xs_pack, W1, b1r, W2, b2r):
    def ein(p):
        def f(b, be_ref):
            return (jnp.maximum(be_ref[b * P + p], 0), 0, 0)
        return f

    grid_spec = pltpu.PrefetchScalarGridSpec(
        num_scalar_prefetch=1,
        grid=(NB // P,),
        in_specs=[
            pl.BlockSpec((P * BM, H // 2), lambda b, be_ref: (b, 0)),
            *[pl.BlockSpec((1, H, I), ein(p)) for p in range(P)],
            *[pl.BlockSpec((1, 1, I), ein(p)) for p in range(P)],
            *[pl.BlockSpec((1, I, H), ein(p)) for p in range(P)],
            *[pl.BlockSpec((1, 1, H), ein(p)) for p in range(P)],
        ],
        out_specs=pl.BlockSpec((P * BM, H // 2), lambda b, be_ref: (b, 0)),
    )
    return pl.pallas_call(
        _mlp_body,
        grid_spec=grid_spec,
        out_shape=jax.ShapeDtypeStruct((BUF, H // 2), jnp.int32),
    )(be, xs_pack,
      *([W1] * P), *([b1r] * P), *([W2] * P), *([b2r] * P))


def _reorder_body(ys_hbm, pos_hbm, ya_hbm, yb_hbm, i0buf, i1buf, bufa, bufb,
                  sema, semb):
    c = lax.axis_index("c")
    s = lax.axis_index("s")
    wid = s * 2 + c

    chunk = T // NW
    t0 = wid * chunk
    pltpu.sync_copy(pos_hbm.at[pl.ds(t0, chunk)], i0buf)
    pltpu.sync_copy(pos_hbm.at[pl.ds(T + t0, chunk)], i1buf)
    cpa = pltpu.async_copy(ys_hbm.at[i0buf], bufa, sema)
    cpb = pltpu.async_copy(ys_hbm.at[i1buf], bufb, semb)
    cpa.wait()
    cpb.wait()
    pltpu.sync_copy(bufa, ya_hbm.at[pl.ds(t0, chunk)])
    pltpu.sync_copy(bufb, yb_hbm.at[pl.ds(t0, chunk)])


def _reorder(ys, pos_flat):
    mesh = plsc.VectorSubcoreMesh(core_axis_name="c", subcore_axis_name="s")
    fn = pl.kernel(
        _reorder_body,
        out_type=(
            jax.ShapeDtypeStruct((T, H // 2), jnp.int32),
            jax.ShapeDtypeStruct((T, H // 2), jnp.int32),
        ),
        mesh=mesh,
        scratch_types=[
            pltpu.VMEM((T // NW,), jnp.int32),
            pltpu.VMEM((T // NW,), jnp.int32),
            pltpu.VMEM((T // NW, H // 2), jnp.int32),
            pltpu.VMEM((T // NW, H // 2), jnp.int32),
            pltpu.SemaphoreType.DMA,
            pltpu.SemaphoreType.DMA,
        ],
    )
    return fn(ys, pos_flat)


FB = 512


def _finish_body(ya_ref, yb_ref, w0_ref, w1_ref, out_ref):
    ya = ya_ref[...]
    yb = yb_ref[...]
    wa = w0_ref[...]
    wb = w1_ref[...]
    alo = lax.bitcast_convert_type(ya << 16, jnp.float32)
    ahi = lax.bitcast_convert_type(ya & jnp.int32(-65536), jnp.float32)
    blo = lax.bitcast_convert_type(yb << 16, jnp.float32)
    bhi = lax.bitcast_convert_type(yb & jnp.int32(-65536), jnp.float32)
    out_ref[:, :H // 2] = alo * wa + blo * wb
    out_ref[:, H // 2:] = ahi * wa + bhi * wb


def _finish(ya, yb, w0, w1):
    return pl.pallas_call(
        _finish_body,
        grid=(T // FB,),
        in_specs=[
            pl.BlockSpec((FB, H // 2), lambda i: (i, 0)),
            pl.BlockSpec((FB, H // 2), lambda i: (i, 0)),
            pl.BlockSpec((FB, 1), lambda i: (i, 0)),
            pl.BlockSpec((FB, 1), lambda i: (i, 0)),
        ],
        out_specs=pl.BlockSpec((FB, H), lambda i: (i, 0)),
        out_shape=jax.ShapeDtypeStruct((T, H), jnp.float32),
    )(ya, yb, w0, w1)


def kernel(x, w_gate, W1, b1, W2, b2):
    xf = x.reshape(T, H)
    w_all, pos, be = _gate_route(xf, w_gate)
    pos_flat = pos.reshape(A)
    lo = lax.bitcast_convert_type(
        xf[:, :H // 2].astype(jnp.bfloat16), jnp.uint16).astype(jnp.int32)
    hi = lax.bitcast_convert_type(
        xf[:, H // 2:].astype(jnp.bfloat16), jnp.uint16).astype(jnp.int32)
    x_pack = lo | (hi << 16)
    xs_pack = _dispatch(x_pack, pos_flat)
    ys = _mlp(be.reshape(NB), xs_pack,
              W1.astype(jnp.bfloat16), b1.reshape(E, 1, I),
              W2.astype(jnp.bfloat16), b2.reshape(E, 1, H))
    ya, yb = _reorder(ys, pos_flat)
    out = _finish(ya, yb, w_all[:T], w_all[T:])
    return out.reshape(x.shape)


# speedup vs baseline: 2.7713x; 1.0685x over previous
import jax
import jax.numpy as jnp
from jax import lax
from jax.experimental import pallas as pl
from jax.experimental.pallas import tpu as pltpu
from jax.experimental.pallas import tpu_sc as plsc

T = 2048
H = 1024
E = 64
K = 2
I = 64
A = T * K
BM = 64
BUF = 8192
NB = BUF // BM
P = 4
NW = 32
TB = 256
GB = 512


def _gate_route_body(x_ref, wg_ref, wall_ref, pos_ref, be_ref, xp_ref, idx_sc):
    step = pl.program_id(0)

    @pl.when(step < T // GB)
    def _gate():
        xb = x_ref[...]
        logits = jnp.dot(xb, wg_ref[...], preferred_element_type=jnp.float32)
        m = jnp.max(logits, axis=-1, keepdims=True)
        ex = jnp.exp(logits - m)
        raw = ex / jnp.sum(ex, axis=-1, keepdims=True)
        lane = lax.broadcasted_iota(jnp.int32, raw.shape, 1)
        p1 = jnp.max(raw, axis=-1, keepdims=True)
        a1 = jnp.min(jnp.where(raw == p1, lane, E), axis=-1, keepdims=True)
        masked = jnp.where(lane == a1, -1.0, raw)
        p2 = jnp.max(masked, axis=-1, keepdims=True)
        a2 = jnp.min(jnp.where(masked == p2, lane, E), axis=-1, keepdims=True)
        e2 = jnp.exp(p2 - p1)
        w0 = 1.0 / (1.0 + e2)
        r0 = step * GB
        xlo = lax.bitcast_convert_type(
            xb[:, :H // 2].astype(jnp.bfloat16), jnp.uint16).astype(jnp.int32)
        xhi = lax.bitcast_convert_type(
            xb[:, H // 2:].astype(jnp.bfloat16), jnp.uint16).astype(jnp.int32)
        xp_ref[...] = xlo | (xhi << 16)
        idx_sc[pl.ds(r0, GB), :] = a1.astype(jnp.int32)
        idx_sc[pl.ds(T + r0, GB), :] = a2.astype(jnp.int32)
        wall_ref[pl.ds(r0, GB), :] = w0
        wall_ref[pl.ds(T + r0, GB), :] = 1.0 - w0

    @pl.when(step == T // GB)
    def _route():
        lane = lax.broadcasted_iota(jnp.int32, (TB, E), 1)
        nch = A // TB
        ohs = []
        run = jnp.zeros((1, E), jnp.float32)
        coffs = []
        for c in range(nch):
            ohc = (idx_sc[pl.ds(c * TB, TB), :] == lane).astype(jnp.float32)
            ohs.append(ohc)
            coffs.append(run)
            run = run + jnp.sum(ohc, axis=0, keepdims=True)
        counts = run
        padded = jnp.ceil(counts / BM) * BM
        li = lax.broadcasted_iota(jnp.int32, (E, E), 0)
        lj = lax.broadcasted_iota(jnp.int32, (E, E), 1)
        ustrict = (li < lj).astype(jnp.float32)
        offs = jnp.dot(padded, ustrict, preferred_element_type=jnp.float32)

        bstart = (lax.broadcasted_iota(jnp.int32, (NB, E), 0) * BM).astype(
            jnp.float32)
        lane_e = lax.broadcasted_iota(jnp.int32, (NB, E), 1)
        sel = (bstart >= offs) & (bstart < offs + padded)
        be = jnp.sum(jnp.where(sel, lane_e, 0), axis=1, keepdims=True)
        hit = jnp.sum(sel.astype(jnp.int32), axis=1, keepdims=True) > 0
        be_ref[...] = jnp.where(hit, be, -1).astype(jnp.int32)

        ci = lax.broadcasted_iota(jnp.int32, (TB, TB), 0)
        cj = lax.broadcasted_iota(jnp.int32, (TB, TB), 1)
        lstrict = (cj < ci).astype(jnp.float32)
        for c in range(nch):
            prior = jnp.dot(lstrict, ohs[c], preferred_element_type=jnp.float32)
            posf = jnp.sum(ohs[c] * (prior + coffs[c] + offs), axis=1,
                           keepdims=True)
            pos_ref[pl.ds(c * TB, TB), :] = posf.astype(jnp.int32)


def _gate_route(xf, w_gate):
    nsteps = T // GB + 1
    return pl.pallas_call(
        _gate_route_body,
        grid=(nsteps,),
        in_specs=[
            pl.BlockSpec((GB, H), lambda i: (jnp.minimum(i, T // GB - 1), 0)),
            pl.BlockSpec((H, E), lambda i: (0, 0)),
        ],
        out_specs=[
            pl.BlockSpec((A, 1), lambda i: (0, 0)),
            pl.BlockSpec((A, 1), lambda i: (0, 0)),
            pl.BlockSpec((NB, 1), lambda i: (0, 0)),
            pl.BlockSpec((GB, H // 2), lambda i: (jnp.minimum(i, T // GB - 1), 0)),
        ],
        out_shape=[
            jax.ShapeDtypeStruct((A, 1), jnp.float32),
            jax.ShapeDtypeStruct((A, 1), jnp.int32),
            jax.ShapeDtypeStruct((NB, 1), jnp.int32),
            jax.ShapeDtypeStruct((T, H // 2), jnp.int32),
        ],
        scratch_shapes=[pltpu.VMEM((A, 1), jnp.int32)],
    )(xf, w_gate)


def _dispatch_body(x_hbm, pos_hbm, xs_hbm, rowbuf, posbuf, sem):
    c = lax.axis_index("c")
    s = lax.axis_index("s")
    wid = s * 2 + c

    rows_per = A // NW
    chunk = 64
    for j in range(rows_per // chunk):
        a0 = wid * rows_per + j * chunk
        t0 = lax.rem(a0, T)
        pltpu.sync_copy(x_hbm.at[pl.ds(t0, chunk)], rowbuf)
        pltpu.sync_copy(pos_hbm.at[pl.ds(a0, chunk)], posbuf)
        pltpu.async_copy(rowbuf, xs_hbm.at[posbuf], sem).wait()


def _dispatch(x_pack, pos_flat):
    mesh = plsc.VectorSubcoreMesh(core_axis_name="c", subcore_axis_name="s")
    fn = pl.kernel(
        _dispatch_body,
        out_type=jax.ShapeDtypeStruct((BUF, H // 2), jnp.int32),
        mesh=mesh,
        scratch_types=[
            pltpu.VMEM((64, H // 2), jnp.int32),
            pltpu.VMEM((64,), jnp.int32),
            pltpu.SemaphoreType.DMA,
        ],
    )
    return fn(x_pack, pos_flat)


def _mlp_body(be_ref, xs_ref, w1_ref, b1_ref, w2_ref, b2_ref, ys_ref):
    b = pl.program_id(0)
    xi = xs_ref[...]

    es = [jnp.maximum(be_ref[b * P + p], 0) for p in range(P)]
    hs = []
    for p in range(P):
        xp = xi[p * BM:(p + 1) * BM, :]
        lo = lax.bitcast_convert_type(xp << 16, jnp.float32)
        hi = lax.bitcast_convert_type(xp & jnp.int32(-65536), jnp.float32)
        w1 = w1_ref[es[p]]
        h = jnp.dot(lo.astype(jnp.bfloat16), w1[:H // 2, :],
                    preferred_element_type=jnp.float32)
        h = h + jnp.dot(hi.astype(jnp.bfloat16), w1[H // 2:, :],
                        preferred_element_type=jnp.float32)
        hs.append(h + b1_ref[es[p]])
    gs = [0.5 * h * (1.0 + lax.erf(h * 0.7071067811865476)) for h in hs]
    for p in range(P):
        y = jnp.dot(gs[p].astype(jnp.bfloat16), w2_ref[es[p]],
                    preferred_element_type=jnp.float32)
        y = y + b2_ref[es[p]]
        ylo = lax.bitcast_convert_type(
            y[:, :H // 2].astype(jnp.bfloat16), jnp.uint16).astype(jnp.int32)
        yhi = lax.bitcast_convert_type(
            y[:, H // 2:].astype(jnp.bfloat16), jnp.uint16).astype(jnp.int32)
        ys_ref[pl.ds(p * BM, BM), :] = ylo | (yhi << 16)


def _mlp(be, xs_pack, W1, b1r, W2, b2r):
    grid_spec = pltpu.PrefetchScalarGridSpec(
        num_scalar_prefetch=1,
        grid=(NB // P,),
        in_specs=[
            pl.BlockSpec((P * BM, H // 2), lambda b, be_ref: (b, 0)),
            pl.BlockSpec((E, H, I), lambda b, be_ref: (0, 0, 0)),
            pl.BlockSpec((E, 1, I), lambda b, be_ref: (0, 0, 0)),
            pl.BlockSpec((E, I, H), lambda b, be_ref: (0, 0, 0)),
            pl.BlockSpec((E, 1, H), lambda b, be_ref: (0, 0, 0)),
        ],
        out_specs=pl.BlockSpec((P * BM, H // 2), lambda b, be_ref: (b, 0)),
    )
    return pl.pallas_call(
        _mlp_body,
        grid_spec=grid_spec,
        out_shape=jax.ShapeDtypeStruct((BUF, H // 2), jnp.int32),
    )(be, xs_pack, W1, b1r, W2, b2r)


def _reorder_body(ys_hbm, pos_hbm, ya_hbm, yb_hbm, i0buf, i1buf, bufa, bufb,
                  sema, semb):
    c = lax.axis_index("c")
    s = lax.axis_index("s")
    wid = s * 2 + c

    chunk = T // NW
    t0 = wid * chunk
    pltpu.sync_copy(pos_hbm.at[pl.ds(t0, chunk)], i0buf)
    pltpu.sync_copy(pos_hbm.at[pl.ds(T + t0, chunk)], i1buf)
    cpa = pltpu.async_copy(ys_hbm.at[i0buf], bufa, sema)
    cpb = pltpu.async_copy(ys_hbm.at[i1buf], bufb, semb)
    cpa.wait()
    cpb.wait()
    pltpu.sync_copy(bufa, ya_hbm.at[pl.ds(t0, chunk)])
    pltpu.sync_copy(bufb, yb_hbm.at[pl.ds(t0, chunk)])


def _reorder(ys, pos_flat):
    mesh = plsc.VectorSubcoreMesh(core_axis_name="c", subcore_axis_name="s")
    fn = pl.kernel(
        _reorder_body,
        out_type=(
            jax.ShapeDtypeStruct((T, H // 2), jnp.int32),
            jax.ShapeDtypeStruct((T, H // 2), jnp.int32),
        ),
        mesh=mesh,
        scratch_types=[
            pltpu.VMEM((T // NW,), jnp.int32),
            pltpu.VMEM((T // NW,), jnp.int32),
            pltpu.VMEM((T // NW, H // 2), jnp.int32),
            pltpu.VMEM((T // NW, H // 2), jnp.int32),
            pltpu.SemaphoreType.DMA,
            pltpu.SemaphoreType.DMA,
        ],
    )
    return fn(ys, pos_flat)


FB = 512


def _finish_body(ya_ref, yb_ref, w0_ref, w1_ref, out_ref):
    ya = ya_ref[...]
    yb = yb_ref[...]
    wa = w0_ref[...]
    wb = w1_ref[...]
    alo = lax.bitcast_convert_type(ya << 16, jnp.float32)
    ahi = lax.bitcast_convert_type(ya & jnp.int32(-65536), jnp.float32)
    blo = lax.bitcast_convert_type(yb << 16, jnp.float32)
    bhi = lax.bitcast_convert_type(yb & jnp.int32(-65536), jnp.float32)
    out_ref[:, :H // 2] = alo * wa + blo * wb
    out_ref[:, H // 2:] = ahi * wa + bhi * wb


def _finish(ya, yb, w0, w1):
    return pl.pallas_call(
        _finish_body,
        grid=(T // FB,),
        in_specs=[
            pl.BlockSpec((FB, H // 2), lambda i: (i, 0)),
            pl.BlockSpec((FB, H // 2), lambda i: (i, 0)),
            pl.BlockSpec((FB, 1), lambda i: (i, 0)),
            pl.BlockSpec((FB, 1), lambda i: (i, 0)),
        ],
        out_specs=pl.BlockSpec((FB, H), lambda i: (i, 0)),
        out_shape=jax.ShapeDtypeStruct((T, H), jnp.float32),
    )(ya, yb, w0, w1)


def kernel(x, w_gate, W1, b1, W2, b2):
    xf = x.reshape(T, H)
    w_all, pos, be, x_pack = _gate_route(xf, w_gate)
    pos_flat = pos.reshape(A)
    xs_pack = _dispatch(x_pack, pos_flat)
    ys = _mlp(be.reshape(NB), xs_pack,
              W1.astype(jnp.bfloat16), b1.reshape(E, 1, I),
              W2.astype(jnp.bfloat16), b2.reshape(E, 1, H))
    ya, yb = _reorder(ys, pos_flat)
    out = _finish(ya, yb, w_all[:T], w_all[T:])
    return out.reshape(x.shape)


# speedup vs baseline: 2.8835x; 1.0405x over previous
import jax
import jax.numpy as jnp
from jax import lax
from jax.experimental import pallas as pl
from jax.experimental.pallas import tpu as pltpu
from jax.experimental.pallas import tpu_sc as plsc

T = 2048
H = 1024
E = 64
K = 2
I = 64
A = T * K
BM = 64
BUF = 8192
NB = BUF // BM
P = 4
NW = 32
TB = 256
GB = 512


def _gate_route_body(x_ref, wg_ref, wall_ref, pos_ref, be_ref, xp_ref, idx_sc):
    step = pl.program_id(0)

    @pl.when(step < T // GB)
    def _gate():
        xb = x_ref[...]
        logits = jnp.dot(xb, wg_ref[...], preferred_element_type=jnp.float32)
        m = jnp.max(logits, axis=-1, keepdims=True)
        ex = jnp.exp(logits - m)
        raw = ex / jnp.sum(ex, axis=-1, keepdims=True)
        lane = lax.broadcasted_iota(jnp.int32, raw.shape, 1)
        p1 = jnp.max(raw, axis=-1, keepdims=True)
        a1 = jnp.min(jnp.where(raw == p1, lane, E), axis=-1, keepdims=True)
        masked = jnp.where(lane == a1, -1.0, raw)
        p2 = jnp.max(masked, axis=-1, keepdims=True)
        a2 = jnp.min(jnp.where(masked == p2, lane, E), axis=-1, keepdims=True)
        e2 = jnp.exp(p2 - p1)
        w0 = 1.0 / (1.0 + e2)
        r0 = step * GB
        xlo = lax.bitcast_convert_type(
            xb[:, :H // 2].astype(jnp.bfloat16), jnp.uint16).astype(jnp.int32)
        xhi = lax.bitcast_convert_type(
            xb[:, H // 2:].astype(jnp.bfloat16), jnp.uint16).astype(jnp.int32)
        xp_ref[...] = xlo | (xhi << 16)
        idx_sc[pl.ds(r0, GB), :] = a1.astype(jnp.int32)
        idx_sc[pl.ds(T + r0, GB), :] = a2.astype(jnp.int32)
        wall_ref[pl.ds(r0, GB), :] = w0
        wall_ref[pl.ds(T + r0, GB), :] = 1.0 - w0

    @pl.when(step == T // GB)
    def _route():
        lane = lax.broadcasted_iota(jnp.int32, (TB, E), 1)
        nch = A // TB
        ohs = []
        run = jnp.zeros((1, E), jnp.float32)
        coffs = []
        for c in range(nch):
            ohc = (idx_sc[pl.ds(c * TB, TB), :] == lane).astype(jnp.float32)
            ohs.append(ohc)
            coffs.append(run)
            run = run + jnp.sum(ohc, axis=0, keepdims=True)
        counts = run
        padded = jnp.ceil(counts / BM) * BM
        li = lax.broadcasted_iota(jnp.int32, (E, E), 0)
        lj = lax.broadcasted_iota(jnp.int32, (E, E), 1)
        ustrict = (li < lj).astype(jnp.float32)
        offs = jnp.dot(padded, ustrict, preferred_element_type=jnp.float32)

        bstart = (lax.broadcasted_iota(jnp.int32, (NB, E), 0) * BM).astype(
            jnp.float32)
        lane_e = lax.broadcasted_iota(jnp.int32, (NB, E), 1)
        sel = (bstart >= offs) & (bstart < offs + padded)
        be = jnp.sum(jnp.where(sel, lane_e, 0), axis=1, keepdims=True)
        hit = jnp.sum(sel.astype(jnp.int32), axis=1, keepdims=True) > 0
        be_ref[...] = jnp.where(hit, be, -1).astype(jnp.int32)

        ci = lax.broadcasted_iota(jnp.int32, (TB, TB), 0)
        cj = lax.broadcasted_iota(jnp.int32, (TB, TB), 1)
        lstrict = (cj < ci).astype(jnp.float32)
        for c in range(nch):
            prior = jnp.dot(lstrict, ohs[c], preferred_element_type=jnp.float32)
            posf = jnp.sum(ohs[c] * (prior + coffs[c] + offs), axis=1,
                           keepdims=True)
            pos_ref[pl.ds(c * TB, TB), :] = posf.astype(jnp.int32)


def _gate_route(xf, w_gate):
    nsteps = T // GB + 1
    return pl.pallas_call(
        _gate_route_body,
        grid=(nsteps,),
        in_specs=[
            pl.BlockSpec((GB, H), lambda i: (jnp.minimum(i, T // GB - 1), 0)),
            pl.BlockSpec((H, E), lambda i: (0, 0)),
        ],
        out_specs=[
            pl.BlockSpec((A, 1), lambda i: (0, 0)),
            pl.BlockSpec((A, 1), lambda i: (0, 0)),
            pl.BlockSpec((NB, 1), lambda i: (0, 0)),
            pl.BlockSpec((GB, H // 2), lambda i: (jnp.minimum(i, T // GB - 1), 0)),
        ],
        out_shape=[
            jax.ShapeDtypeStruct((A, 1), jnp.float32),
            jax.ShapeDtypeStruct((A, 1), jnp.int32),
            jax.ShapeDtypeStruct((NB, 1), jnp.int32),
            jax.ShapeDtypeStruct((T, H // 2), jnp.int32),
        ],
        scratch_shapes=[pltpu.VMEM((A, 1), jnp.int32)],
    )(xf, w_gate)


def _dispatch_body(x_hbm, pos_hbm, xs_hbm, rowbuf, posbuf, sem):
    c = lax.axis_index("c")
    s = lax.axis_index("s")
    wid = s * 2 + c

    rows_per = A // NW
    chunk = 64
    for j in range(rows_per // chunk):
        a0 = wid * rows_per + j * chunk
        t0 = lax.rem(a0, T)
        pltpu.sync_copy(x_hbm.at[pl.ds(t0, chunk)], rowbuf)
        pltpu.sync_copy(pos_hbm.at[pl.ds(a0, chunk)], posbuf)
        pltpu.async_copy(rowbuf, xs_hbm.at[posbuf], sem).wait()


def _dispatch(x_pack, pos_flat):
    mesh = plsc.VectorSubcoreMesh(core_axis_name="c", subcore_axis_name="s")
    fn = pl.kernel(
        _dispatch_body,
        out_type=jax.ShapeDtypeStruct((BUF, H // 2), jnp.int32),
        mesh=mesh,
        scratch_types=[
            pltpu.VMEM((64, H // 2), jnp.int32),
            pltpu.VMEM((64,), jnp.int32),
            pltpu.SemaphoreType.DMA,
        ],
    )
    return fn(x_pack, pos_flat)


def _mlp_body(be_ref, xs_ref, w1_ref, b1_ref, w2_ref, b2_ref, ys_ref):
    b = pl.program_id(0)
    xi = xs_ref[...]

    es = [jnp.maximum(be_ref[b * P + p], 0) for p in range(P)]
    hs = []
    for p in range(P):
        xp = xi[p * BM:(p + 1) * BM, :]
        lo = lax.bitcast_convert_type(xp << 16, jnp.float32)
        hi = lax.bitcast_convert_type(xp & jnp.int32(-65536), jnp.float32)
        w1 = w1_ref[es[p]].astype(jnp.bfloat16)
        h = jnp.dot(lo.astype(jnp.bfloat16), w1[:H // 2, :],
                    preferred_element_type=jnp.float32)
        h = h + jnp.dot(hi.astype(jnp.bfloat16), w1[H // 2:, :],
                        preferred_element_type=jnp.float32)
        hs.append(h + b1_ref[es[p]])
    gs = [0.5 * h * (1.0 + lax.erf(h * 0.7071067811865476)) for h in hs]
    for p in range(P):
        y = jnp.dot(gs[p].astype(jnp.bfloat16),
                    w2_ref[es[p]].astype(jnp.bfloat16),
                    preferred_element_type=jnp.float32)
        y = y + b2_ref[es[p]]
        ylo = lax.bitcast_convert_type(
            y[:, :H // 2].astype(jnp.bfloat16), jnp.uint16).astype(jnp.int32)
        yhi = lax.bitcast_convert_type(
            y[:, H // 2:].astype(jnp.bfloat16), jnp.uint16).astype(jnp.int32)
        ys_ref[pl.ds(p * BM, BM), :] = ylo | (yhi << 16)


def _mlp(be, xs_pack, W1, b1r, W2, b2r):
    grid_spec = pltpu.PrefetchScalarGridSpec(
        num_scalar_prefetch=1,
        grid=(NB // P,),
        in_specs=[
            pl.BlockSpec((P * BM, H // 2), lambda b, be_ref: (b, 0)),
            pl.BlockSpec((E, H, I), lambda b, be_ref: (0, 0, 0)),
            pl.BlockSpec((E, 1, I), lambda b, be_ref: (0, 0, 0)),
            pl.BlockSpec((E, I, H), lambda b, be_ref: (0, 0, 0)),
            pl.BlockSpec((E, 1, H), lambda b, be_ref: (0, 0, 0)),
        ],
        out_specs=pl.BlockSpec((P * BM, H // 2), lambda b, be_ref: (b, 0)),
    )
    return pl.pallas_call(
        _mlp_body,
        grid_spec=grid_spec,
        out_shape=jax.ShapeDtypeStruct((BUF, H // 2), jnp.int32),
    )(be, xs_pack, W1, b1r, W2, b2r)


def _reorder_body(ys_hbm, pos_hbm, ya_hbm, yb_hbm, i0buf, i1buf, bufa, bufb,
                  sema, semb):
    c = lax.axis_index("c")
    s = lax.axis_index("s")
    wid = s * 2 + c

    chunk = T // NW
    t0 = wid * chunk
    pltpu.sync_copy(pos_hbm.at[pl.ds(t0, chunk)], i0buf)
    pltpu.sync_copy(pos_hbm.at[pl.ds(T + t0, chunk)], i1buf)
    cpa = pltpu.async_copy(ys_hbm.at[i0buf], bufa, sema)
    cpb = pltpu.async_copy(ys_hbm.at[i1buf], bufb, semb)
    cpa.wait()
    cpb.wait()
    pltpu.sync_copy(bufa, ya_hbm.at[pl.ds(t0, chunk)])
    pltpu.sync_copy(bufb, yb_hbm.at[pl.ds(t0, chunk)])


def _reorder(ys, pos_flat):
    mesh = plsc.VectorSubcoreMesh(core_axis_name="c", subcore_axis_name="s")
    fn = pl.kernel(
        _reorder_body,
        out_type=(
            jax.ShapeDtypeStruct((T, H // 2), jnp.int32),
            jax.ShapeDtypeStruct((T, H // 2), jnp.int32),
        ),
        mesh=mesh,
        scratch_types=[
            pltpu.VMEM((T // NW,), jnp.int32),
            pltpu.VMEM((T // NW,), jnp.int32),
            pltpu.VMEM((T // NW, H // 2), jnp.int32),
            pltpu.VMEM((T // NW, H // 2), jnp.int32),
            pltpu.SemaphoreType.DMA,
            pltpu.SemaphoreType.DMA,
        ],
    )
    return fn(ys, pos_flat)


FB = 512


def _finish_body(ya_ref, yb_ref, w0_ref, w1_ref, out_ref):
    ya = ya_ref[...]
    yb = yb_ref[...]
    wa = w0_ref[...]
    wb = w1_ref[...]
    alo = lax.bitcast_convert_type(ya << 16, jnp.float32)
    ahi = lax.bitcast_convert_type(ya & jnp.int32(-65536), jnp.float32)
    blo = lax.bitcast_convert_type(yb << 16, jnp.float32)
    bhi = lax.bitcast_convert_type(yb & jnp.int32(-65536), jnp.float32)
    out_ref[:, :H // 2] = alo * wa + blo * wb
    out_ref[:, H // 2:] = ahi * wa + bhi * wb


def _finish(ya, yb, w0, w1):
    return pl.pallas_call(
        _finish_body,
        grid=(T // FB,),
        in_specs=[
            pl.BlockSpec((FB, H // 2), lambda i: (i, 0)),
            pl.BlockSpec((FB, H // 2), lambda i: (i, 0)),
            pl.BlockSpec((FB, 1), lambda i: (i, 0)),
            pl.BlockSpec((FB, 1), lambda i: (i, 0)),
        ],
        out_specs=pl.BlockSpec((FB, H), lambda i: (i, 0)),
        out_shape=jax.ShapeDtypeStruct((T, H), jnp.float32),
    )(ya, yb, w0, w1)


def kernel(x, w_gate, W1, b1, W2, b2):
    xf = x.reshape(T, H)
    w_all, pos, be, x_pack = _gate_route(xf, w_gate)
    pos_flat = pos.reshape(A)
    xs_pack = _dispatch(x_pack, pos_flat)
    ys = _mlp(be.reshape(NB), xs_pack,
              W1, b1.reshape(E, 1, I), W2, b2.reshape(E, 1, H))
    ya, yb = _reorder(ys, pos_flat)
    out = _finish(ya, yb, w_all[:T], w_all[T:])
    return out.reshape(x.shape)


# speedup vs baseline: 3.0800x; 1.0681x over previous
import jax
import jax.numpy as jnp
from jax import lax
from jax.experimental import pallas as pl
from jax.experimental.pallas import tpu as pltpu
from jax.experimental.pallas import tpu_sc as plsc

T = 2048
H = 1024
E = 64
K = 2
I = 64
A = T * K
BM = 64
BUF = 8192
NB = BUF // BM
P = 8
NW = 32
TB = 256
GB = 512


def _gate_route_body(x_ref, wg_ref, wall_ref, pos_ref, be_ref, xp_ref, idx_sc):
    step = pl.program_id(0)

    @pl.when(step < T // GB)
    def _gate():
        xb = x_ref[...]
        logits = jnp.dot(xb, wg_ref[...], preferred_element_type=jnp.float32)
        m = jnp.max(logits, axis=-1, keepdims=True)
        ex = jnp.exp(logits - m)
        raw = ex / jnp.sum(ex, axis=-1, keepdims=True)
        lane = lax.broadcasted_iota(jnp.int32, raw.shape, 1)
        p1 = jnp.max(raw, axis=-1, keepdims=True)
        a1 = jnp.min(jnp.where(raw == p1, lane, E), axis=-1, keepdims=True)
        masked = jnp.where(lane == a1, -1.0, raw)
        p2 = jnp.max(masked, axis=-1, keepdims=True)
        a2 = jnp.min(jnp.where(masked == p2, lane, E), axis=-1, keepdims=True)
        e2 = jnp.exp(p2 - p1)
        w0 = 1.0 / (1.0 + e2)
        r0 = step * GB
        xlo = lax.bitcast_convert_type(
            xb[:, :H // 2].astype(jnp.bfloat16), jnp.uint16).astype(jnp.int32)
        xhi = lax.bitcast_convert_type(
            xb[:, H // 2:].astype(jnp.bfloat16), jnp.uint16).astype(jnp.int32)
        xp_ref[...] = xlo | (xhi << 16)
        idx_sc[pl.ds(r0, GB), :] = a1.astype(jnp.int32)
        idx_sc[pl.ds(T + r0, GB), :] = a2.astype(jnp.int32)
        wall_ref[pl.ds(r0, GB), :] = w0
        wall_ref[pl.ds(T + r0, GB), :] = 1.0 - w0

    @pl.when(step == T // GB)
    def _route():
        lane = lax.broadcasted_iota(jnp.int32, (TB, E), 1)
        nch = A // TB
        ohs = []
        run = jnp.zeros((1, E), jnp.float32)
        coffs = []
        for c in range(nch):
            ohc = (idx_sc[pl.ds(c * TB, TB), :] == lane).astype(jnp.float32)
            ohs.append(ohc)
            coffs.append(run)
            run = run + jnp.sum(ohc, axis=0, keepdims=True)
        counts = run
        padded = jnp.ceil(counts / BM) * BM
        li = lax.broadcasted_iota(jnp.int32, (E, E), 0)
        lj = lax.broadcasted_iota(jnp.int32, (E, E), 1)
        ustrict = (li < lj).astype(jnp.float32)
        offs = jnp.dot(padded, ustrict, preferred_element_type=jnp.float32)

        bstart = (lax.broadcasted_iota(jnp.int32, (NB, E), 0) * BM).astype(
            jnp.float32)
        lane_e = lax.broadcasted_iota(jnp.int32, (NB, E), 1)
        sel = (bstart >= offs) & (bstart < offs + padded)
        be = jnp.sum(jnp.where(sel, lane_e, 0), axis=1, keepdims=True)
        hit = jnp.sum(sel.astype(jnp.int32), axis=1, keepdims=True) > 0
        be_ref[...] = jnp.where(hit, be, -1).astype(jnp.int32)

        ci = lax.broadcasted_iota(jnp.int32, (TB, TB), 0)
        cj = lax.broadcasted_iota(jnp.int32, (TB, TB), 1)
        lstrict = (cj < ci).astype(jnp.float32)
        for c in range(nch):
            prior = jnp.dot(lstrict, ohs[c], preferred_element_type=jnp.float32)
            posf = jnp.sum(ohs[c] * (prior + coffs[c] + offs), axis=1,
                           keepdims=True)
            pos_ref[pl.ds(c * TB, TB), :] = posf.astype(jnp.int32)


def _gate_route(xf, w_gate):
    nsteps = T // GB + 1
    return pl.pallas_call(
        _gate_route_body,
        grid=(nsteps,),
        in_specs=[
            pl.BlockSpec((GB, H), lambda i: (jnp.minimum(i, T // GB - 1), 0)),
            pl.BlockSpec((H, E), lambda i: (0, 0)),
        ],
        out_specs=[
            pl.BlockSpec((A, 1), lambda i: (0, 0)),
            pl.BlockSpec((A, 1), lambda i: (0, 0)),
            pl.BlockSpec((NB, 1), lambda i: (0, 0)),
            pl.BlockSpec((GB, H // 2), lambda i: (jnp.minimum(i, T // GB - 1), 0)),
        ],
        out_shape=[
            jax.ShapeDtypeStruct((A, 1), jnp.float32),
            jax.ShapeDtypeStruct((A, 1), jnp.int32),
            jax.ShapeDtypeStruct((NB, 1), jnp.int32),
            jax.ShapeDtypeStruct((T, H // 2), jnp.int32),
        ],
        scratch_shapes=[pltpu.VMEM((A, 1), jnp.int32)],
    )(xf, w_gate)


def _dispatch_body(x_hbm, pos_hbm, xs_hbm, rowbuf, posbuf, sem):
    c = lax.axis_index("c")
    s = lax.axis_index("s")
    wid = s * 2 + c

    rows_per = A // NW
    chunk = 64
    for j in range(rows_per // chunk):
        a0 = wid * rows_per + j * chunk
        t0 = lax.rem(a0, T)
        pltpu.sync_copy(x_hbm.at[pl.ds(t0, chunk)], rowbuf)
        pltpu.sync_copy(pos_hbm.at[pl.ds(a0, chunk)], posbuf)
        pltpu.async_copy(rowbuf, xs_hbm.at[posbuf], sem).wait()


def _dispatch(x_pack, pos_flat):
    mesh = plsc.VectorSubcoreMesh(core_axis_name="c", subcore_axis_name="s")
    fn = pl.kernel(
        _dispatch_body,
        out_type=jax.ShapeDtypeStruct((BUF, H // 2), jnp.int32),
        mesh=mesh,
        scratch_types=[
            pltpu.VMEM((64, H // 2), jnp.int32),
            pltpu.VMEM((64,), jnp.int32),
            pltpu.SemaphoreType.DMA,
        ],
    )
    return fn(x_pack, pos_flat)


def _mlp_body(be_ref, xs_ref, w1_ref, b1_ref, w2_ref, b2_ref, ys_ref):
    b = pl.program_id(0)
    xi = xs_ref[...]

    es = [jnp.maximum(be_ref[b * P + p], 0) for p in range(P)]
    hs = []
    for p in range(P):
        xp = xi[p * BM:(p + 1) * BM, :]
        lo = lax.bitcast_convert_type(xp << 16, jnp.float32)
        hi = lax.bitcast_convert_type(xp & jnp.int32(-65536), jnp.float32)
        w1 = w1_ref[es[p]].astype(jnp.bfloat16)
        h = jnp.dot(lo.astype(jnp.bfloat16), w1[:H // 2, :],
                    preferred_element_type=jnp.float32)
        h = h + jnp.dot(hi.astype(jnp.bfloat16), w1[H // 2:, :],
                        preferred_element_type=jnp.float32)
        hs.append(h + b1_ref[es[p]])
    gs = [0.5 * h * (1.0 + lax.erf(h * 0.7071067811865476)) for h in hs]
    for p in range(P):
        y = jnp.dot(gs[p].astype(jnp.bfloat16),
                    w2_ref[es[p]].astype(jnp.bfloat16),
                    preferred_element_type=jnp.float32)
        y = y + b2_ref[es[p]]
        ylo = lax.bitcast_convert_type(
            y[:, :H // 2].astype(jnp.bfloat16), jnp.uint16).astype(jnp.int32)
        yhi = lax.bitcast_convert_type(
            y[:, H // 2:].astype(jnp.bfloat16), jnp.uint16).astype(jnp.int32)
        ys_ref[pl.ds(p * BM, BM), :] = ylo | (yhi << 16)


def _mlp(be, xs_pack, W1, b1r, W2, b2r):
    grid_spec = pltpu.PrefetchScalarGridSpec(
        num_scalar_prefetch=1,
        grid=(NB // P,),
        in_specs=[
            pl.BlockSpec((P * BM, H // 2), lambda b, be_ref: (b, 0)),
            pl.BlockSpec((E, H, I), lambda b, be_ref: (0, 0, 0)),
            pl.BlockSpec((E, 1, I), lambda b, be_ref: (0, 0, 0)),
            pl.BlockSpec((E, I, H), lambda b, be_ref: (0, 0, 0)),
            pl.BlockSpec((E, 1, H), lambda b, be_ref: (0, 0, 0)),
        ],
        out_specs=pl.BlockSpec((P * BM, H // 2), lambda b, be_ref: (b, 0)),
    )
    return pl.pallas_call(
        _mlp_body,
        grid_spec=grid_spec,
        out_shape=jax.ShapeDtypeStruct((BUF, H // 2), jnp.int32),
    )(be, xs_pack, W1, b1r, W2, b2r)


def _reorder_body(ys_hbm, pos_hbm, ya_hbm, yb_hbm, i0buf, i1buf, bufa, bufb,
                  sema, semb):
    c = lax.axis_index("c")
    s = lax.axis_index("s")
    wid = s * 2 + c

    chunk = T // NW
    t0 = wid * chunk
    pltpu.sync_copy(pos_hbm.at[pl.ds(t0, chunk)], i0buf)
    pltpu.sync_copy(pos_hbm.at[pl.ds(T + t0, chunk)], i1buf)
    cpa = pltpu.async_copy(ys_hbm.at[i0buf], bufa, sema)
    cpb = pltpu.async_copy(ys_hbm.at[i1buf], bufb, semb)
    cpa.wait()
    cpb.wait()
    pltpu.sync_copy(bufa, ya_hbm.at[pl.ds(t0, chunk)])
    pltpu.sync_copy(bufb, yb_hbm.at[pl.ds(t0, chunk)])


def _reorder(ys, pos_flat):
    mesh = plsc.VectorSubcoreMesh(core_axis_name="c", subcore_axis_name="s")
    fn = pl.kernel(
        _reorder_body,
        out_type=(
            jax.ShapeDtypeStruct((T, H // 2), jnp.int32),
            jax.ShapeDtypeStruct((T, H // 2), jnp.int32),
        ),
        mesh=mesh,
        scratch_types=[
            pltpu.VMEM((T // NW,), jnp.int32),
            pltpu.VMEM((T // NW,), jnp.int32),
            pltpu.VMEM((T // NW, H // 2), jnp.int32),
            pltpu.VMEM((T // NW, H // 2), jnp.int32),
            pltpu.SemaphoreType.DMA,
            pltpu.SemaphoreType.DMA,
        ],
    )
    return fn(ys, pos_flat)


FB = 512


def _finish_body(ya_ref, yb_ref, w0_ref, w1_ref, out_ref):
    ya = ya_ref[...]
    yb = yb_ref[...]
    wa = w0_ref[...]
    wb = w1_ref[...]
    alo = lax.bitcast_convert_type(ya << 16, jnp.float32)
    ahi = lax.bitcast_convert_type(ya & jnp.int32(-65536), jnp.float32)
    blo = lax.bitcast_convert_type(yb << 16, jnp.float32)
    bhi = lax.bitcast_convert_type(yb & jnp.int32(-65536), jnp.float32)
    out_ref[:, :H // 2] = alo * wa + blo * wb
    out_ref[:, H // 2:] = ahi * wa + bhi * wb


def _finish(ya, yb, w0, w1):
    return pl.pallas_call(
        _finish_body,
        grid=(T // FB,),
        in_specs=[
            pl.BlockSpec((FB, H // 2), lambda i: (i, 0)),
            pl.BlockSpec((FB, H // 2), lambda i: (i, 0)),
            pl.BlockSpec((FB, 1), lambda i: (i, 0)),
            pl.BlockSpec((FB, 1), lambda i: (i, 0)),
        ],
        out_specs=pl.BlockSpec((FB, H), lambda i: (i, 0)),
        out_shape=jax.ShapeDtypeStruct((T, H), jnp.float32),
    )(ya, yb, w0, w1)


def kernel(x, w_gate, W1, b1, W2, b2):
    xf = x.reshape(T, H)
    w_all, pos, be, x_pack = _gate_route(xf, w_gate)
    pos_flat = pos.reshape(A)
    xs_pack = _dispatch(x_pack, pos_flat)
    ys = _mlp(be.reshape(NB), xs_pack,
              W1, b1.reshape(E, 1, I), W2, b2.reshape(E, 1, H))
    ya, yb = _reorder(ys, pos_flat)
    out = _finish(ya, yb, w_all[:T], w_all[T:])
    return out.reshape(x.shape)


# speedup vs baseline: 3.1703x; 1.0293x over previous
import jax
import jax.numpy as jnp
from jax import lax
from jax.experimental import pallas as pl
from jax.experimental.pallas import tpu as pltpu
from jax.experimental.pallas import tpu_sc as plsc

T = 2048
H = 1024
E = 64
K = 2
I = 64
A = T * K
BM = 64
BUF = 8192
NB = BUF // BM
P = 16
NW = 32
TB = 256
GB = 512


def _gate_route_body(x_ref, wg_ref, wall_ref, pos_ref, be_ref, xp_ref, idx_sc):
    step = pl.program_id(0)

    @pl.when(step < T // GB)
    def _gate():
        xb = x_ref[...]
        logits = jnp.dot(xb, wg_ref[...], preferred_element_type=jnp.float32)
        m = jnp.max(logits, axis=-1, keepdims=True)
        ex = jnp.exp(logits - m)
        raw = ex / jnp.sum(ex, axis=-1, keepdims=True)
        lane = lax.broadcasted_iota(jnp.int32, raw.shape, 1)
        p1 = jnp.max(raw, axis=-1, keepdims=True)
        a1 = jnp.min(jnp.where(raw == p1, lane, E), axis=-1, keepdims=True)
        masked = jnp.where(lane == a1, -1.0, raw)
        p2 = jnp.max(masked, axis=-1, keepdims=True)
        a2 = jnp.min(jnp.where(masked == p2, lane, E), axis=-1, keepdims=True)
        e2 = jnp.exp(p2 - p1)
        w0 = 1.0 / (1.0 + e2)
        r0 = step * GB
        xlo = lax.bitcast_convert_type(
            xb[:, :H // 2].astype(jnp.bfloat16), jnp.uint16).astype(jnp.int32)
        xhi = lax.bitcast_convert_type(
            xb[:, H // 2:].astype(jnp.bfloat16), jnp.uint16).astype(jnp.int32)
        xp_ref[...] = xlo | (xhi << 16)
        idx_sc[pl.ds(r0, GB), :] = a1.astype(jnp.int32)
        idx_sc[pl.ds(T + r0, GB), :] = a2.astype(jnp.int32)
        wall_ref[pl.ds(r0, GB), :] = w0
        wall_ref[pl.ds(T + r0, GB), :] = 1.0 - w0

    @pl.when(step == T // GB)
    def _route():
        lane = lax.broadcasted_iota(jnp.int32, (TB, E), 1)
        nch = A // TB
        ohs = []
        run = jnp.zeros((1, E), jnp.float32)
        coffs = []
        for c in range(nch):
            ohc = (idx_sc[pl.ds(c * TB, TB), :] == lane).astype(jnp.float32)
            ohs.append(ohc)
            coffs.append(run)
            run = run + jnp.sum(ohc, axis=0, keepdims=True)
        counts = run
        padded = jnp.ceil(counts / BM) * BM
        li = lax.broadcasted_iota(jnp.int32, (E, E), 0)
        lj = lax.broadcasted_iota(jnp.int32, (E, E), 1)
        ustrict = (li < lj).astype(jnp.float32)
        offs = jnp.dot(padded, ustrict, preferred_element_type=jnp.float32)

        bstart = (lax.broadcasted_iota(jnp.int32, (NB, E), 0) * BM).astype(
            jnp.float32)
        lane_e = lax.broadcasted_iota(jnp.int32, (NB, E), 1)
        sel = (bstart >= offs) & (bstart < offs + padded)
        be = jnp.sum(jnp.where(sel, lane_e, 0), axis=1, keepdims=True)
        hit = jnp.sum(sel.astype(jnp.int32), axis=1, keepdims=True) > 0
        be_ref[...] = jnp.where(hit, be, -1).astype(jnp.int32)

        ci = lax.broadcasted_iota(jnp.int32, (TB, TB), 0)
        cj = lax.broadcasted_iota(jnp.int32, (TB, TB), 1)
        lstrict = (cj < ci).astype(jnp.float32)
        for c in range(nch):
            prior = jnp.dot(lstrict, ohs[c], preferred_element_type=jnp.float32)
            posf = jnp.sum(ohs[c] * (prior + coffs[c] + offs), axis=1,
                           keepdims=True)
            pos_ref[pl.ds(c * TB, TB), :] = posf.astype(jnp.int32)


def _gate_route(xf, w_gate):
    nsteps = T // GB + 1
    return pl.pallas_call(
        _gate_route_body,
        grid=(nsteps,),
        in_specs=[
            pl.BlockSpec((GB, H), lambda i: (jnp.minimum(i, T // GB - 1), 0)),
            pl.BlockSpec((H, E), lambda i: (0, 0)),
        ],
        out_specs=[
            pl.BlockSpec((A, 1), lambda i: (0, 0)),
            pl.BlockSpec((A, 1), lambda i: (0, 0)),
            pl.BlockSpec((NB, 1), lambda i: (0, 0)),
            pl.BlockSpec((GB, H // 2), lambda i: (jnp.minimum(i, T // GB - 1), 0)),
        ],
        out_shape=[
            jax.ShapeDtypeStruct((A, 1), jnp.float32),
            jax.ShapeDtypeStruct((A, 1), jnp.int32),
            jax.ShapeDtypeStruct((NB, 1), jnp.int32),
            jax.ShapeDtypeStruct((T, H // 2), jnp.int32),
        ],
        scratch_shapes=[pltpu.VMEM((A, 1), jnp.int32)],
    )(xf, w_gate)


def _dispatch_body(x_hbm, pos_hbm, xs_hbm, rowbuf, posbuf, sem):
    c = lax.axis_index("c")
    s = lax.axis_index("s")
    wid = s * 2 + c

    rows_per = A // NW
    chunk = 64
    for j in range(rows_per // chunk):
        a0 = wid * rows_per + j * chunk
        t0 = lax.rem(a0, T)
        pltpu.sync_copy(x_hbm.at[pl.ds(t0, chunk)], rowbuf)
        pltpu.sync_copy(pos_hbm.at[pl.ds(a0, chunk)], posbuf)
        pltpu.async_copy(rowbuf, xs_hbm.at[posbuf], sem).wait()


def _dispatch(x_pack, pos_flat):
    mesh = plsc.VectorSubcoreMesh(core_axis_name="c", subcore_axis_name="s")
    fn = pl.kernel(
        _dispatch_body,
        out_type=jax.ShapeDtypeStruct((BUF, H // 2), jnp.int32),
        mesh=mesh,
        scratch_types=[
            pltpu.VMEM((64, H // 2), jnp.int32),
            pltpu.VMEM((64,), jnp.int32),
            pltpu.SemaphoreType.DMA,
        ],
    )
    return fn(x_pack, pos_flat)


def _mlp_body(be_ref, xs_ref, w1_ref, b1_ref, w2_ref, b2_ref, ys_ref):
    b = pl.program_id(0)
    xi = xs_ref[...]

    es = [jnp.maximum(be_ref[b * P + p], 0) for p in range(P)]
    hs = []
    for p in range(P):
        xp = xi[p * BM:(p + 1) * BM, :]
        lo = lax.bitcast_convert_type(xp << 16, jnp.float32)
        hi = lax.bitcast_convert_type(xp & jnp.int32(-65536), jnp.float32)
        w1 = w1_ref[es[p]].astype(jnp.bfloat16)
        h = jnp.dot(lo.astype(jnp.bfloat16), w1[:H // 2, :],
                    preferred_element_type=jnp.float32)
        h = h + jnp.dot(hi.astype(jnp.bfloat16), w1[H // 2:, :],
                        preferred_element_type=jnp.float32)
        hs.append(h + b1_ref[es[p]])
    gs = [0.5 * h * (1.0 + lax.erf(h * 0.7071067811865476)) for h in hs]
    for p in range(P):
        y = jnp.dot(gs[p].astype(jnp.bfloat16),
                    w2_ref[es[p]].astype(jnp.bfloat16),
                    preferred_element_type=jnp.float32)
        y = y + b2_ref[es[p]]
        ylo = lax.bitcast_convert_type(
            y[:, :H // 2].astype(jnp.bfloat16), jnp.uint16).astype(jnp.int32)
        yhi = lax.bitcast_convert_type(
            y[:, H // 2:].astype(jnp.bfloat16), jnp.uint16).astype(jnp.int32)
        ys_ref[pl.ds(p * BM, BM), :] = ylo | (yhi << 16)


def _mlp(be, xs_pack, W1, b1r, W2, b2r):
    grid_spec = pltpu.PrefetchScalarGridSpec(
        num_scalar_prefetch=1,
        grid=(NB // P,),
        in_specs=[
            pl.BlockSpec((P * BM, H // 2), lambda b, be_ref: (b, 0)),
            pl.BlockSpec((E, H, I), lambda b, be_ref: (0, 0, 0)),
            pl.BlockSpec((E, 1, I), lambda b, be_ref: (0, 0, 0)),
            pl.BlockSpec((E, I, H), lambda b, be_ref: (0, 0, 0)),
            pl.BlockSpec((E, 1, H), lambda b, be_ref: (0, 0, 0)),
        ],
        out_specs=pl.BlockSpec((P * BM, H // 2), lambda b, be_ref: (b, 0)),
    )
    return pl.pallas_call(
        _mlp_body,
        grid_spec=grid_spec,
        out_shape=jax.ShapeDtypeStruct((BUF, H // 2), jnp.int32),
    )(be, xs_pack, W1, b1r, W2, b2r)


def _reorder_body(ys_hbm, pos_hbm, ya_hbm, yb_hbm, i0buf, i1buf, bufa, bufb,
                  sema, semb):
    c = lax.axis_index("c")
    s = lax.axis_index("s")
    wid = s * 2 + c

    chunk = T // NW
    t0 = wid * chunk
    pltpu.sync_copy(pos_hbm.at[pl.ds(t0, chunk)], i0buf)
    pltpu.sync_copy(pos_hbm.at[pl.ds(T + t0, chunk)], i1buf)
    cpa = pltpu.async_copy(ys_hbm.at[i0buf], bufa, sema)
    cpb = pltpu.async_copy(ys_hbm.at[i1buf], bufb, semb)
    cpa.wait()
    cpb.wait()
    pltpu.sync_copy(bufa, ya_hbm.at[pl.ds(t0, chunk)])
    pltpu.sync_copy(bufb, yb_hbm.at[pl.ds(t0, chunk)])


def _reorder(ys, pos_flat):
    mesh = plsc.VectorSubcoreMesh(core_axis_name="c", subcore_axis_name="s")
    fn = pl.kernel(
        _reorder_body,
        out_type=(
            jax.ShapeDtypeStruct((T, H // 2), jnp.int32),
            jax.ShapeDtypeStruct((T, H // 2), jnp.int32),
        ),
        mesh=mesh,
        scratch_types=[
            pltpu.VMEM((T // NW,), jnp.int32),
            pltpu.VMEM((T // NW,), jnp.int32),
            pltpu.VMEM((T // NW, H // 2), jnp.int32),
            pltpu.VMEM((T // NW, H // 2), jnp.int32),
            pltpu.SemaphoreType.DMA,
            pltpu.SemaphoreType.DMA,
        ],
    )
    return fn(ys, pos_flat)


FB = 512


def _finish_body(ya_ref, yb_ref, w0_ref, w1_ref, out_ref):
    ya = ya_ref[...]
    yb = yb_ref[...]
    wa = w0_ref[...]
    wb = w1_ref[...]
    alo = lax.bitcast_convert_type(ya << 16, jnp.float32)
    ahi = lax.bitcast_convert_type(ya & jnp.int32(-65536), jnp.float32)
    blo = lax.bitcast_convert_type(yb << 16, jnp.float32)
    bhi = lax.bitcast_convert_type(yb & jnp.int32(-65536), jnp.float32)
    out_ref[:, :H // 2] = alo * wa + blo * wb
    out_ref[:, H // 2:] = ahi * wa + bhi * wb


def _finish(ya, yb, w0, w1):
    return pl.pallas_call(
        _finish_body,
        grid=(T // FB,),
        in_specs=[
            pl.BlockSpec((FB, H // 2), lambda i: (i, 0)),
            pl.BlockSpec((FB, H // 2), lambda i: (i, 0)),
            pl.BlockSpec((FB, 1), lambda i: (i, 0)),
            pl.BlockSpec((FB, 1), lambda i: (i, 0)),
        ],
        out_specs=pl.BlockSpec((FB, H), lambda i: (i, 0)),
        out_shape=jax.ShapeDtypeStruct((T, H), jnp.float32),
    )(ya, yb, w0, w1)


def kernel(x, w_gate, W1, b1, W2, b2):
    xf = x.reshape(T, H)
    w_all, pos, be, x_pack = _gate_route(xf, w_gate)
    pos_flat = pos.reshape(A)
    xs_pack = _dispatch(x_pack, pos_flat)
    ys = _mlp(be.reshape(NB), xs_pack,
              W1, b1.reshape(E, 1, I), W2, b2.reshape(E, 1, H))
    ya, yb = _reorder(ys, pos_flat)
    out = _finish(ya, yb, w_all[:T], w_all[T:])
    return out.reshape(x.shape)
